# baseline (speedup 1.0000x reference)
import functools
import math

import jax
import jax.numpy as jnp
from jax import lax
from jax.experimental import pallas as pl
from jax.experimental.pallas import tpu as pltpu
from jax.experimental.pallas import tpu_sc as plsc

N = 10000
E = 320000
NUM_ELEM = 10
H = 128
NB = 8
RH = 64
RMAX = 5.0

NC = 2
NS = 16
NW = NC * NS
EPW = E // NW
C = 80
NCHUNK = EPW // C
NPS = 624
ZR = 48
POSW = 16
C1 = 64
CT = 16
CF = 128

f32 = jnp.float32

_mesh = plsc.VectorSubcoreMesh(
    core_axis_name="c", subcore_axis_name="s", num_cores=NC, num_subcores=NS)
_sc_params = pltpu.CompilerParams(needs_layout_passes=False)


def _wid():
    return lax.axis_index("s") * NC + lax.axis_index("c")


def _zero_shared(sh_ref, zbuf, sid):
    zz = jnp.zeros((16,), f32)

    def zrow(i, _):
        for j in range(H // 16):
            zbuf[i, pl.ds(j * 16, 16)] = zz
        return 0

    lax.fori_loop(0, ZR, zrow, 0)
    for q in range(NPS // ZR):
        pltpu.sync_copy(zbuf.at[pl.ds(0, ZR)],
                        sh_ref.at[pl.ds(sid * NPS + q * ZR, ZR)])

    @pl.when(sid == NS - 1)
    def _():
        pltpu.sync_copy(zbuf.at[pl.ds(0, 16)],
                        sh_ref.at[pl.ds(NS * NPS, 16)])


def _copy_out_shared(sh_ref, out_hbm, cid, sid):
    pltpu.sync_copy(sh_ref.at[pl.ds(sid * NPS, NPS)],
                    out_hbm.at[cid, pl.ds(sid * NPS, NPS)])

    @pl.when(sid == NS - 1)
    def _():
        pltpu.sync_copy(sh_ref.at[pl.ds(NS * NPS, 16)],
                        out_hbm.at[cid, pl.ds(NS * NPS, 16)])


def _sc_vec_body(pos_hbm, src_hbm, dst_hbm, vec_hbm, pos_v, idx_s, idx_d,
                 vbuf):
    wid = _wid()
    pltpu.sync_copy(pos_hbm, pos_v)
    zz = jnp.zeros((16,), f32)

    def zrow(i, _):
        vbuf[i, :] = zz
        return 0

    lax.fori_loop(0, C, zrow, 0)

    def chunk(t, _):
        base = pl.multiple_of(wid * EPW + t * C, 8)
        pltpu.sync_copy(src_hbm.at[pl.ds(base, C)], idx_s)
        pltpu.sync_copy(dst_hbm.at[pl.ds(base, C)], idx_d)

        def group(g, _):
            rows16 = lax.iota(jnp.int32, 16) + g * 16
            s16 = idx_s[pl.ds(g * 16, 16)] * 4
            d16 = idx_d[pl.ds(g * 16, 16)] * 4
            for c in range(3):
                ps = plsc.load_gather(pos_v, [s16 + c])
                pd = plsc.load_gather(pos_v, [d16 + c])
                plsc.store_scatter(vbuf, [rows16, jnp.full((16,), c, jnp.int32)],
                                   pd - ps)
            return 0

        lax.fori_loop(0, C // 16, group, 0)
        pltpu.sync_copy(vbuf, vec_hbm.at[pl.ds(base, C)])
        return 0

    lax.fori_loop(0, NCHUNK, chunk, 0)


def _sc_vec(pos_flat, src, dst):
    return pl.kernel(
        _sc_vec_body,
        out_type=jax.ShapeDtypeStruct((E, POSW), f32),
        mesh=_mesh,
        compiler_params=_sc_params,
        scratch_types=[
            pltpu.VMEM((N * 4,), f32),
            pltpu.VMEM((C,), jnp.int32),
            pltpu.VMEM((C,), jnp.int32),
            pltpu.VMEM((C, POSW), f32),
        ],
    )(pos_flat, src, dst)


def _sc_edge_fwd_body(h_hbm, r_hbm, src_hbm, dst_hbm, agg_hbm,
                      idx_s0, idx_s1, idx_d0, idx_d1, rows0, rows1,
                      rbuf0, rbuf1, agg_sh, sg0, sg1, sr0, sr1, ss0, ss1):
    cid = lax.axis_index("c")
    sid = lax.axis_index("s")
    wid = sid * NC + cid
    idx_s = (idx_s0, idx_s1)
    idx_d = (idx_d0, idx_d1)
    rows = (rows0, rows1)
    rbuf = (rbuf0, rbuf1)
    sg = (sg0, sg1)
    sr = (sr0, sr1)
    ss = (ss0, ss1)
    _zero_shared(agg_sh, rbuf0, sid)
    plsc.subcore_barrier()

    def load(t, b):
        @pl.when(t >= 2)
        def _():
            pltpu.make_async_copy(rbuf[b], agg_sh.at[idx_d[b]], ss[b]).wait()

        base = pl.multiple_of(wid * EPW + t * C, 8)
        pltpu.sync_copy(src_hbm.at[pl.ds(base, C)], idx_s[b])
        pltpu.sync_copy(dst_hbm.at[pl.ds(base, C)], idx_d[b])
        pltpu.async_copy(h_hbm.at[idx_s[b]], rows[b], sg[b])
        pltpu.async_copy(r_hbm.at[pl.ds(base, C)], rbuf[b], sr[b])

    def process(t, b):
        base = pl.multiple_of(wid * EPW + t * C, 8)
        pltpu.make_async_copy(h_hbm.at[idx_s[b]], rows[b], sg[b]).wait()
        pltpu.make_async_copy(r_hbm.at[pl.ds(base, C)], rbuf[b], sr[b]).wait()

        def row(i, _):
            for j in range(H // 16):
                sl = pl.ds(j * 16, 16)
                rbuf[b][i, sl] = rows[b][i, sl] * rbuf[b][i, sl]
            return 0

        lax.fori_loop(0, C, row, 0)
        pltpu.async_copy(rbuf[b], agg_sh.at[idx_d[b]], ss[b], add=True)

    load(0, 0)

    def pair(t2, _):
        for b in range(2):
            t = 2 * t2 + b
            load(t + 1, 1 - b)
            process(t, b)
        return 0

    lax.fori_loop(0, (NCHUNK - 1) // 2, pair, 0)
    process(NCHUNK - 1, 0)
    pltpu.make_async_copy(rbuf[0], agg_sh.at[idx_d[0]], ss[0]).wait()
    pltpu.make_async_copy(rbuf[1], agg_sh.at[idx_d[1]], ss[1]).wait()
    plsc.subcore_barrier()
    _copy_out_shared(agg_sh, agg_hbm, cid, sid)


def _sc_edge_fwd(h, R, src, dst):
    return pl.kernel(
        _sc_edge_fwd_body,
        out_type=jax.ShapeDtypeStruct((NC, N, H), f32),
        mesh=_mesh,
        compiler_params=_sc_params,
        scratch_types=[
            pltpu.VMEM((C,), jnp.int32),
            pltpu.VMEM((C,), jnp.int32),
            pltpu.VMEM((C,), jnp.int32),
            pltpu.VMEM((C,), jnp.int32),
            pltpu.VMEM((C, H), f32),
            pltpu.VMEM((C, H), f32),
            pltpu.VMEM((C, H), f32),
            pltpu.VMEM((C, H), f32),
            pltpu.VMEM_SHARED((N, H), f32),
            pltpu.SemaphoreType.DMA,
            pltpu.SemaphoreType.DMA,
            pltpu.SemaphoreType.DMA,
            pltpu.SemaphoreType.DMA,
            pltpu.SemaphoreType.DMA,
            pltpu.SemaphoreType.DMA,
        ],
    )(h, R, src, dst)


def _sc_edge_bwd1_body(dagg_hbm, h_hbm, r_hbm, src_hbm, dst_hbm, dr_hbm,
                       dh_hbm, idx_s0, idx_s1, idx_d0, idx_d1, idx_st, idx_dt,
                       drows0, drows1, hrows0, hrows1, rbuf0, rbuf1, dh_sh,
                       sa0, sa1, sh0, sh1, sr0, sr1, sw0, sw1, ss0, ss1):
    cid = lax.axis_index("c")
    sid = lax.axis_index("s")
    wid = sid * NC + cid
    idx_s = (idx_s0, idx_s1)
    idx_d = (idx_d0, idx_d1)
    drows = (drows0, drows1)
    hrows = (hrows0, hrows1)
    rbuf = (rbuf0, rbuf1)
    sa = (sa0, sa1)
    sh = (sh0, sh1)
    sr = (sr0, sr1)
    sw = (sw0, sw1)
    ss = (ss0, ss1)
    _zero_shared(dh_sh, rbuf0, sid)
    plsc.subcore_barrier()
    NCH1 = EPW // C1

    def load(t, b):
        @pl.when(t >= 2)
        def _():
            base0 = pl.multiple_of(wid * EPW + (t - 2) * C1, 8)
            pltpu.make_async_copy(hrows[b], dr_hbm.at[pl.ds(base0, C1)],
                                  sw[b]).wait()
            pltpu.make_async_copy(rbuf[b], dh_sh.at[idx_s[b]], ss[b]).wait()

        base = pl.multiple_of(wid * EPW + t * C1, 8)
        pltpu.sync_copy(src_hbm.at[pl.ds(base, C1)], idx_s[b])
        pltpu.sync_copy(dst_hbm.at[pl.ds(base, C1)], idx_d[b])
        pltpu.async_copy(dagg_hbm.at[idx_d[b]], drows[b], sa[b])
        pltpu.async_copy(h_hbm.at[idx_s[b]], hrows[b], sh[b])
        pltpu.async_copy(r_hbm.at[pl.ds(base, C1)], rbuf[b], sr[b])

    def process(t, b):
        base = pl.multiple_of(wid * EPW + t * C1, 8)
        pltpu.make_async_copy(dagg_hbm.at[idx_d[b]], drows[b], sa[b]).wait()
        pltpu.make_async_copy(h_hbm.at[idx_s[b]], hrows[b], sh[b]).wait()
        pltpu.make_async_copy(r_hbm.at[pl.ds(base, C1)], rbuf[b], sr[b]).wait()

        def row(i, _):
            for j in range(H // 16):
                sl = pl.ds(j * 16, 16)
                dm = drows[b][i, sl]
                hrows[b][i, sl] = dm * hrows[b][i, sl]
                rbuf[b][i, sl] = dm * rbuf[b][i, sl]
            return 0

        lax.fori_loop(0, C1, row, 0)
        pltpu.async_copy(hrows[b], dr_hbm.at[pl.ds(base, C1)], sw[b])
        pltpu.async_copy(rbuf[b], dh_sh.at[idx_s[b]], ss[b], add=True)

    load(0, 0)

    def pair(t2, _):
        load(2 * t2 + 1, 1)
        process(2 * t2, 0)

        @pl.when(t2 < (NCH1 // 2) - 1)
        def _():
            load(2 * t2 + 2, 0)

        process(2 * t2 + 1, 1)
        return 0

    lax.fori_loop(0, NCH1 // 2, pair, 0)
    for b in range(2):
        basef = pl.multiple_of(wid * EPW + (NCH1 - 2 + b) * C1, 8)
        pltpu.make_async_copy(hrows[b], dr_hbm.at[pl.ds(basef, C1)],
                              sw[b]).wait()
        pltpu.make_async_copy(rbuf[b], dh_sh.at[idx_s[b]], ss[b]).wait()

    base = pl.multiple_of(wid * EPW + NCH1 * C1, 8)
    pltpu.sync_copy(src_hbm.at[pl.ds(base, CT)], idx_st)
    pltpu.sync_copy(dst_hbm.at[pl.ds(base, CT)], idx_dt)
    pltpu.async_copy(dagg_hbm.at[idx_dt], drows0.at[pl.ds(0, CT)], sa0).wait()
    pltpu.async_copy(h_hbm.at[idx_st], hrows0.at[pl.ds(0, CT)], sh0).wait()
    pltpu.async_copy(r_hbm.at[pl.ds(base, CT)], rbuf0.at[pl.ds(0, CT)],
                     sr0).wait()

    def trow(i, _):
        for j in range(H // 16):
            sl = pl.ds(j * 16, 16)
            dm = drows0[i, sl]
            hrows0[i, sl] = dm * hrows0[i, sl]
            rbuf0[i, sl] = dm * rbuf0[i, sl]
        return 0

    lax.fori_loop(0, CT, trow, 0)
    pltpu.sync_copy(hrows0.at[pl.ds(0, CT)], dr_hbm.at[pl.ds(base, CT)])
    pltpu.sync_copy(rbuf0.at[pl.ds(0, CT)], dh_sh.at[idx_st], add=True)

    plsc.subcore_barrier()
    _copy_out_shared(dh_sh, dh_hbm, cid, sid)


def _sc_edge_bwd1(dagg, h, R, src, dst):
    return pl.kernel(
        _sc_edge_bwd1_body,
        out_type=(jax.ShapeDtypeStruct((E, H), f32),
                  jax.ShapeDtypeStruct((NC, N, H), f32)),
        mesh=_mesh,
        compiler_params=_sc_params,
        scratch_types=[
            pltpu.VMEM((C1,), jnp.int32),
            pltpu.VMEM((C1,), jnp.int32),
            pltpu.VMEM((C1,), jnp.int32),
            pltpu.VMEM((C1,), jnp.int32),
            pltpu.VMEM((CT,), jnp.int32),
            pltpu.VMEM((CT,), jnp.int32),
            pltpu.VMEM((C1, H), f32),
            pltpu.VMEM((C1, H), f32),
            pltpu.VMEM((C1, H), f32),
            pltpu.VMEM((C1, H), f32),
            pltpu.VMEM((C1, H), f32),
            pltpu.VMEM((C1, H), f32),
            pltpu.VMEM_SHARED((N, H), f32),
            pltpu.SemaphoreType.DMA,
            pltpu.SemaphoreType.DMA,
            pltpu.SemaphoreType.DMA,
            pltpu.SemaphoreType.DMA,
            pltpu.SemaphoreType.DMA,
            pltpu.SemaphoreType.DMA,
            pltpu.SemaphoreType.DMA,
            pltpu.SemaphoreType.DMA,
            pltpu.SemaphoreType.DMA,
            pltpu.SemaphoreType.DMA,
        ],
    )(dagg, h, R, src, dst)


def _sc_edge_bwd0_body(dagg_hbm, h_hbm, src_hbm, dst_hbm, dr_hbm,
                       idx_s0, idx_s1, idx_d0, idx_d1, drows0, drows1,
                       hrows0, hrows1, sa0, sa1, sh0, sh1, sw0, sw1):
    wid = _wid()
    idx_s = (idx_s0, idx_s1)
    idx_d = (idx_d0, idx_d1)
    drows = (drows0, drows1)
    hrows = (hrows0, hrows1)
    sa = (sa0, sa1)
    sh = (sh0, sh1)
    sw = (sw0, sw1)

    def load(t, b):
        @pl.when(t >= 2)
        def _():
            base0 = pl.multiple_of(wid * EPW + (t - 2) * C, 8)
            pltpu.make_async_copy(hrows[b], dr_hbm.at[pl.ds(base0, C)],
                                  sw[b]).wait()

        base = pl.multiple_of(wid * EPW + t * C, 8)
        pltpu.sync_copy(src_hbm.at[pl.ds(base, C)], idx_s[b])
        pltpu.sync_copy(dst_hbm.at[pl.ds(base, C)], idx_d[b])
        pltpu.async_copy(dagg_hbm.at[idx_d[b]], drows[b], sa[b])
        pltpu.async_copy(h_hbm.at[idx_s[b]], hrows[b], sh[b])

    def process(t, b):
        base = pl.multiple_of(wid * EPW + t * C, 8)
        pltpu.make_async_copy(dagg_hbm.at[idx_d[b]], drows[b], sa[b]).wait()
        pltpu.make_async_copy(h_hbm.at[idx_s[b]], hrows[b], sh[b]).wait()

        def row(i, _):
            for j in range(H // 16):
                sl = pl.ds(j * 16, 16)
                hrows[b][i, sl] = drows[b][i, sl] * hrows[b][i, sl]
            return 0

        lax.fori_loop(0, C, row, 0)
        pltpu.async_copy(hrows[b], dr_hbm.at[pl.ds(base, C)], sw[b])

    load(0, 0)

    def pair(t2, _):
        for b in range(2):
            t = 2 * t2 + b
            load(t + 1, 1 - b)
            process(t, b)
        return 0

    lax.fori_loop(0, (NCHUNK - 1) // 2, pair, 0)
    process(NCHUNK - 1, 0)
    for b in range(2):
        basef = pl.multiple_of(wid * EPW + (NCHUNK - 2 + b) * C, 8)
        pltpu.make_async_copy(hrows[b], dr_hbm.at[pl.ds(basef, C)],
                              sw[b]).wait()


def _sc_edge_bwd0(dagg, h, src, dst):
    return pl.kernel(
        _sc_edge_bwd0_body,
        out_type=jax.ShapeDtypeStruct((E, H), f32),
        mesh=_mesh,
        compiler_params=_sc_params,
        scratch_types=[
            pltpu.VMEM((C,), jnp.int32),
            pltpu.VMEM((C,), jnp.int32),
            pltpu.VMEM((C,), jnp.int32),
            pltpu.VMEM((C,), jnp.int32),
            pltpu.VMEM((C, H), f32),
            pltpu.VMEM((C, H), f32),
            pltpu.VMEM((C, H), f32),
            pltpu.VMEM((C, H), f32),
            pltpu.SemaphoreType.DMA,
            pltpu.SemaphoreType.DMA,
            pltpu.SemaphoreType.DMA,
            pltpu.SemaphoreType.DMA,
            pltpu.SemaphoreType.DMA,
            pltpu.SemaphoreType.DMA,
        ],
    )(dagg, h, src, dst)


def _sc_forces_body(dv_hbm, src_hbm, dst_hbm, fp_hbm, idx_s, idx_d, dvb,
                    acc, sv):
    wid = _wid()
    zz = jnp.zeros((16,), f32)

    def zrow(k, _):
        acc[pl.ds(k * 16, 16)] = zz
        return 0

    lax.fori_loop(0, (N * 4) // 16, zrow, 0)

    def do_chunk(q):
        base = pl.multiple_of(q * CF, 8)
        pltpu.sync_copy(src_hbm.at[pl.ds(base, CF)], idx_s)
        pltpu.sync_copy(dst_hbm.at[pl.ds(base, CF)], idx_d)
        pltpu.async_copy(dv_hbm.at[:, pl.ds(base, CF)], dvb, sv).wait()

        def group(g, _):
            s16 = idx_s[pl.ds(g * 16, 16)] * 4
            d16 = idx_d[pl.ds(g * 16, 16)] * 4
            for c in range(3):
                vals = dvb[c, pl.ds(g * 16, 16)]
                plsc.addupdate_scatter(acc, [s16 + c], vals)
                plsc.addupdate_scatter(acc, [d16 + c], -vals)
            return 0

        lax.fori_loop(0, CF // 16, group, 0)

    def chunk(t, _):
        do_chunk(wid + NW * t)
        return 0

    lax.fori_loop(0, (E // CF) // NW, chunk, 0)

    @pl.when(wid < (E // CF) % NW)
    def _():
        do_chunk(((E // CF) // NW) * NW + wid)

    pltpu.sync_copy(acc, fp_hbm.at[pl.ds(wid * (N * 4), N * 4)])


def _sc_forces(dvec_t, src, dst):
    return pl.kernel(
        _sc_forces_body,
        out_type=jax.ShapeDtypeStruct((NW * N * 4,), f32),
        mesh=_mesh,
        compiler_params=_sc_params,
        scratch_types=[
            pltpu.VMEM((CF,), jnp.int32),
            pltpu.VMEM((CF,), jnp.int32),
            pltpu.VMEM((4, CF), f32),
            pltpu.VMEM((N * 4,), f32),
            pltpu.SemaphoreType.DMA,
        ],
    )(dvec_t, src, dst)


BN = 1000
BE = 1280


def _silu_prime(z):
    sg = jax.nn.sigmoid(z)
    return sg * (1.0 + z * (1.0 - sg))


def _embed_body(at_ref, w_ref, o_ref):
    t = at_ref[0, 0, :]
    oh = (t[:, None] == lax.broadcasted_iota(jnp.int32, (1, NUM_ELEM), 1)
          ).astype(f32)
    o_ref[...] = jnp.dot(oh, w_ref[...], preferred_element_type=f32)


def _embed(atom_types, W_emb):
    at3 = atom_types.reshape(N // BN, 1, BN)
    return pl.pallas_call(
        _embed_body,
        grid=(N // BN,),
        in_specs=[
            pl.BlockSpec((1, 1, BN), lambda i: (i, 0, 0)),
            pl.BlockSpec((NUM_ELEM, H), lambda i: (0, 0)),
        ],
        out_specs=pl.BlockSpec((BN, H), lambda i: (i, 0)),
        out_shape=jax.ShapeDtypeStruct((N, H), f32),
    )(at3, W_emb)


def _trig1(r):
    rc = jnp.clip(r, 0.0, RMAX)
    t = (math.pi / RMAX) * rc - (math.pi / 2)
    u = t * t
    s1 = 1.0 + u * (-1.0 / 2 + u * (1.0 / 24 + u * (-1.0 / 720 + u * (
        1.0 / 40320 + u * (-1.0 / 3628800 + u * (1.0 / 479001600))))))
    sp = 1.0 + u * (-1.0 / 6 + u * (1.0 / 120 + u * (-1.0 / 5040 + u * (
        1.0 / 362880 + u * (-1.0 / 39916800)))))
    c1 = -t * sp
    return s1, c1


def _harm_sigma(s1, c1, r):
    kn = math.pi / RMAX
    x = kn * jnp.clip(r, 0.0, RMAX)
    sigma1 = kn * jnp.where(x < 0.8, _sincp(x * x), s1 / jnp.maximum(x, 0.5))
    rs1 = r * s1
    sig, cn = sigma1, c1
    sigs, ccs = [sig], [cn]
    for _ in range(NB - 1):
        sig, cn = sig * c1 + cn * sigma1, cn * c1 - sig * rs1
        sigs.append(sig)
        ccs.append(cn)
    return sigs, ccs


def _stack8(cols):
    return jnp.concatenate([x[:, None] for x in cols], axis=1)


def _sincp(w):
    return 1.0 + w * (-1.0 / 6 + w * (1.0 / 120 + w * (-1.0 / 5040 + w * (
        1.0 / 362880))))


def _radial_fwd_body(vec_ref, w1c_ref, w2bd_ref, rbt_ref, vt_ref,
                     r0_ref, r1_ref):
    vt16 = jnp.transpose(vec_ref[...])
    vt = vt16[:4, :]
    s = vt[0] * vt[0] + vt[1] * vt[1] + vt[2] * vt[2] + 1e-12
    r = jnp.sqrt(s)
    s1, c1 = _trig1(r)
    sigs, _ = _harm_sigma(s1, c1, r)
    env = 0.5 * (c1 + 1.0)
    coef = math.sqrt(2.0 / RMAX) * env
    rbt = jnp.concatenate([(coef * sg)[None, :] for sg in sigs], axis=0)
    rbt_ref[...] = rbt
    vt_ref[...] = vt
    aq = lax.dot_general(rbt, w1c_ref[...], (((0,), (0,)), ((), ())),
                         preferred_element_type=f32)
    s01 = aq * jax.nn.sigmoid(aq)
    r01 = jnp.dot(s01, w2bd_ref[...], preferred_element_type=f32)
    r0_ref[...] = r01[:, :H]
    r1_ref[...] = r01[:, H:]


def _radial_fwd(vec16, W1cat, W2bd):
    return pl.pallas_call(
        _radial_fwd_body,
        grid=(E // BE,),
        in_specs=[
            pl.BlockSpec((BE, POSW), lambda i: (i, 0)),
            pl.BlockSpec((NB, 2 * RH), lambda i: (0, 0)),
            pl.BlockSpec((2 * RH, 2 * H), lambda i: (0, 0)),
        ],
        out_specs=[
            pl.BlockSpec((NB, BE), lambda i: (0, i)),
            pl.BlockSpec((4, BE), lambda i: (0, i)),
            pl.BlockSpec((BE, H), lambda i: (i, 0)),
            pl.BlockSpec((BE, H), lambda i: (i, 0)),
        ],
        out_shape=[
            jax.ShapeDtypeStruct((NB, E), f32),
            jax.ShapeDtypeStruct((4, E), f32),
            jax.ShapeDtypeStruct((E, H), f32),
            jax.ShapeDtypeStruct((E, H), f32),
        ],
    )(vec16, W1cat, W2bd)


def _node_fwd_body(aggp_ref, wu_ref, z_ref, h_ref):
    a = aggp_ref[0] + aggp_ref[1]
    z = jnp.dot(a, wu_ref[...], preferred_element_type=f32)
    z_ref[...] = z
    h_ref[...] = z * jax.nn.sigmoid(z)


def _node_fwd(aggp, Wu):
    return pl.pallas_call(
        _node_fwd_body,
        grid=(N // BN,),
        in_specs=[
            pl.BlockSpec((NC, BN, H), lambda i: (0, i, 0)),
            pl.BlockSpec((H, H), lambda i: (0, 0)),
        ],
        out_specs=[
            pl.BlockSpec((BN, H), lambda i: (i, 0)),
            pl.BlockSpec((BN, H), lambda i: (i, 0)),
        ],
        out_shape=[
            jax.ShapeDtypeStruct((N, H), f32),
            jax.ShapeDtypeStruct((N, H), f32),
        ],
    )(aggp, Wu)


def _top_bwd_body(z_ref, h2_ref, wuT_ref, wo_ref, dagg_ref, e_ref):
    i = pl.program_id(0)
    z = z_ref[...]
    wo = wo_ref[...]
    dz = wo * _silu_prime(z)
    dagg_ref[...] = jnp.dot(dz, wuT_ref[...], preferred_element_type=f32)
    part = jnp.sum(h2_ref[...] * wo)

    @pl.when(i == 0)
    def _():
        e_ref[0, 0] = 0.0

    e_ref[0, 0] += part


def _top_bwd(z1, h2, Wu_1T, wo2d):
    return pl.pallas_call(
        _top_bwd_body,
        grid=(N // BN,),
        in_specs=[
            pl.BlockSpec((BN, H), lambda i: (i, 0)),
            pl.BlockSpec((BN, H), lambda i: (i, 0)),
            pl.BlockSpec((H, H), lambda i: (0, 0)),
            pl.BlockSpec((1, H), lambda i: (0, 0)),
        ],
        out_specs=[
            pl.BlockSpec((BN, H), lambda i: (i, 0)),
            pl.BlockSpec(memory_space=pltpu.SMEM),
        ],
        out_shape=[
            jax.ShapeDtypeStruct((N, H), f32),
            jax.ShapeDtypeStruct((1, 1), f32),
        ],
    )(z1, h2, Wu_1T, wo2d)


def _mid_bwd_body(dhp_ref, z_ref, wuT_ref, dagg_ref):
    dh = dhp_ref[0] + dhp_ref[1]
    z = z_ref[...]
    dagg_ref[...] = jnp.dot(dh * _silu_prime(z), wuT_ref[...],
                            preferred_element_type=f32)


def _mid_bwd(dhp, z0, Wu_0T):
    return pl.pallas_call(
        _mid_bwd_body,
        grid=(N // BN,),
        in_specs=[
            pl.BlockSpec((NC, BN, H), lambda i: (0, i, 0)),
            pl.BlockSpec((BN, H), lambda i: (i, 0)),
            pl.BlockSpec((H, H), lambda i: (0, 0)),
        ],
        out_specs=pl.BlockSpec((BN, H), lambda i: (i, 0)),
        out_shape=jax.ShapeDtypeStruct((N, H), f32),
    )(dhp, z0, Wu_0T)


def _radial_bwd_body(dr0_ref, dr1_ref, rbt_ref, vt_ref, w1c_ref, w2bdT_ref,
                     dv_ref):
    vt = vt_ref[...]
    s = vt[0] * vt[0] + vt[1] * vt[1] + vt[2] * vt[2] + 1e-12
    r = jnp.sqrt(s)
    s1, c1 = _trig1(r)
    sigs, ccs = _harm_sigma(s1, c1, r)
    env = 0.5 * (c1 + 1.0)
    rbt = rbt_ref[...]
    aq = lax.dot_general(rbt, w1c_ref[...], (((0,), (0,)), ((), ())),
                         preferred_element_type=f32)
    dr01 = jnp.concatenate([dr0_ref[...], dr1_ref[...]], axis=1)
    ds01 = jnp.dot(dr01, w2bdT_ref[...], preferred_element_type=f32)
    da01 = ds01 * _silu_prime(aq)
    drbt = lax.dot_general(w1c_ref[...], da01, (((1,), (1,)), ((), ())),
                           preferred_element_type=f32)
    c0 = math.sqrt(2.0 / RMAX)
    rinv = 1.0 / r
    kn = math.pi / RMAX
    envp = jnp.where(r < RMAX, -0.5 * kn * s1, 0.0)
    er = env * rinv
    wmat = jnp.concatenate(
        [(c0 * (er * ((n + 1) * kn * cn - sg) + envp * sg))[None, :]
         for n, (sg, cn) in enumerate(zip(sigs, ccs))], axis=0)
    dr = jnp.sum(drbt * wmat, axis=0)
    dv_ref[...] = (dr * rinv)[None, :] * vt


def _radial_bwd(dR0, dR1, rbt, vec_t, W1cat, W2bdT):
    return pl.pallas_call(
        _radial_bwd_body,
        grid=(E // BE,),
        in_specs=[
            pl.BlockSpec((BE, H), lambda i: (i, 0)),
            pl.BlockSpec((BE, H), lambda i: (i, 0)),
            pl.BlockSpec((NB, BE), lambda i: (0, i)),
            pl.BlockSpec((4, BE), lambda i: (0, i)),
            pl.BlockSpec((NB, 2 * RH), lambda i: (0, 0)),
            pl.BlockSpec((2 * H, 2 * RH), lambda i: (0, 0)),
        ],
        out_specs=pl.BlockSpec((4, BE), lambda i: (0, i)),
        out_shape=jax.ShapeDtypeStruct((4, E), f32),
    )(dR0, dR1, rbt, vec_t, W1cat, W2bdT)


FRB = 4096


def _force_reduce_body(fp_ref, o_ref):
    o_ref[...] = jnp.sum(fp_ref[...], axis=0)


def _force_reduce(fp):
    nblk = (N * 4 + FRB - 1) // FRB
    return pl.pallas_call(
        _force_reduce_body,
        grid=(nblk,),
        in_specs=[pl.BlockSpec((NW, FRB), lambda i: (0, i))],
        out_specs=pl.BlockSpec((FRB,), lambda i: (i,)),
        out_shape=jax.ShapeDtypeStruct((N * 4,), f32),
    )(fp)


def kernel(positions, atom_types, edge_index, batch, W_emb, Wr1_0, Wr2_0,
           Wu_0, Wr1_1, Wr2_1, Wu_1, w_out):
    del batch
    pos_flat = jnp.concatenate(
        [positions, jnp.zeros((N, 1), f32)], axis=1).reshape(N * 4)
    src = edge_index[0]
    dst = edge_index[1]

    W1cat = jnp.concatenate([Wr1_0, Wr1_1], axis=1)
    W2bd = jnp.zeros((2 * RH, 2 * H), f32)
    W2bd = W2bd.at[:RH, :H].set(Wr2_0).at[RH:, H:].set(Wr2_1)
    W2bdT = jnp.zeros((2 * H, 2 * RH), f32)
    W2bdT = W2bdT.at[:H, :RH].set(Wr2_0.T).at[H:, RH:].set(Wr2_1.T)

    h0 = _embed(atom_types, W_emb)
    vec16 = _sc_vec(pos_flat, src, dst)
    rbt, vec_t, R0, R1 = _radial_fwd(vec16, W1cat, W2bd)

    aggp0 = _sc_edge_fwd(h0, R0, src, dst)
    z0, h1 = _node_fwd(aggp0, Wu_0)
    aggp1 = _sc_edge_fwd(h1, R1, src, dst)
    z1, h2 = _node_fwd(aggp1, Wu_1)

    dagg1, e11 = _top_bwd(z1, h2, Wu_1.T, w_out.reshape(1, H))
    dR1, dhp1 = _sc_edge_bwd1(dagg1, h1, R1, src, dst)
    dagg0 = _mid_bwd(dhp1, z0, Wu_0.T)
    dR0 = _sc_edge_bwd0(dagg0, h0, src, dst)

    dvec_t = _radial_bwd(dR0, dR1, rbt, vec_t, W1cat, W2bdT)
    fp = _sc_forces(dvec_t, src, dst).reshape(NW, N * 4)
    forces4 = _force_reduce(fp)

    energy = e11.reshape(1)
    forces = forces4.reshape(N, 4)[:, :3]
    return energy, forces


# speedup vs baseline: 1.0637x; 1.0637x over previous
import functools
import math

import jax
import jax.numpy as jnp
from jax import lax
from jax.experimental import pallas as pl
from jax.experimental.pallas import tpu as pltpu
from jax.experimental.pallas import tpu_sc as plsc

N = 10000
E = 320000
NUM_ELEM = 10
H = 128
NB = 8
RH = 64
RMAX = 5.0

NC = 2
NS = 16
NW = NC * NS
EPW = E // NW
C = 80
NCHUNK = EPW // C
NPS = 624
ZR = 48
POSW = 16
C1 = 64
CT = 16
CF = 128

f32 = jnp.float32

_mesh = plsc.VectorSubcoreMesh(
    core_axis_name="c", subcore_axis_name="s", num_cores=NC, num_subcores=NS)
_sc_params = pltpu.CompilerParams(needs_layout_passes=False)


def _wid():
    return lax.axis_index("s") * NC + lax.axis_index("c")


def _zero_shared(sh_ref, zbuf, sid):
    zz = jnp.zeros((16,), f32)

    def zrow(i, _):
        for j in range(H // 16):
            zbuf[i, pl.ds(j * 16, 16)] = zz
        return 0

    lax.fori_loop(0, ZR, zrow, 0)
    for q in range(NPS // ZR):
        pltpu.sync_copy(zbuf.at[pl.ds(0, ZR)],
                        sh_ref.at[pl.ds(sid * NPS + q * ZR, ZR)])

    @pl.when(sid == NS - 1)
    def _():
        pltpu.sync_copy(zbuf.at[pl.ds(0, 16)],
                        sh_ref.at[pl.ds(NS * NPS, 16)])


def _copy_out_shared(sh_ref, out_hbm, cid, sid):
    pltpu.sync_copy(sh_ref.at[pl.ds(sid * NPS, NPS)],
                    out_hbm.at[cid, pl.ds(sid * NPS, NPS)])

    @pl.when(sid == NS - 1)
    def _():
        pltpu.sync_copy(sh_ref.at[pl.ds(NS * NPS, 16)],
                        out_hbm.at[cid, pl.ds(NS * NPS, 16)])


def _sc_vec_body(pos_hbm, src_hbm, dst_hbm, vec_hbm, pos_v, idx_s0, idx_s1,
                 idx_d0, idx_d1, vbuf0, vbuf1, si0, si1, sj0, sj1, sw0, sw1):
    wid = _wid()
    idx_s = (idx_s0, idx_s1)
    idx_d = (idx_d0, idx_d1)
    vbuf = (vbuf0, vbuf1)
    si = (si0, si1)
    sj = (sj0, sj1)
    sw = (sw0, sw1)
    pltpu.sync_copy(pos_hbm, pos_v)
    zz = jnp.zeros((16,), f32)

    def zrow(i, _):
        vbuf0[i, :] = zz
        vbuf1[i, :] = zz
        return 0

    lax.fori_loop(0, C, zrow, 0)

    def load(t, b):
        base = pl.multiple_of(wid * EPW + t * C, 8)
        pltpu.async_copy(src_hbm.at[pl.ds(base, C)], idx_s[b], si[b])
        pltpu.async_copy(dst_hbm.at[pl.ds(base, C)], idx_d[b], sj[b])

    def process(t, b):
        base = pl.multiple_of(wid * EPW + t * C, 8)
        pltpu.make_async_copy(src_hbm.at[pl.ds(base, C)], idx_s[b],
                              si[b]).wait()
        pltpu.make_async_copy(dst_hbm.at[pl.ds(base, C)], idx_d[b],
                              sj[b]).wait()

        @pl.when(t >= 2)
        def _():
            base0 = pl.multiple_of(wid * EPW + (t - 2) * C, 8)
            pltpu.make_async_copy(vbuf[b], vec_hbm.at[pl.ds(base0, C)],
                                  sw[b]).wait()

        def group(g, _):
            rows16 = lax.iota(jnp.int32, 16) + g * 16
            s16 = idx_s[b][pl.ds(g * 16, 16)] * 4
            d16 = idx_d[b][pl.ds(g * 16, 16)] * 4
            for c in range(3):
                ps = plsc.load_gather(pos_v, [s16 + c])
                pd = plsc.load_gather(pos_v, [d16 + c])
                plsc.store_scatter(vbuf[b],
                                   [rows16, jnp.full((16,), c, jnp.int32)],
                                   pd - ps)
            return 0

        lax.fori_loop(0, C // 16, group, 0)
        pltpu.async_copy(vbuf[b], vec_hbm.at[pl.ds(base, C)], sw[b])

    load(0, 0)

    def pair(t2, _):
        for b in range(2):
            t = 2 * t2 + b
            load(t + 1, 1 - b)
            process(t, b)
        return 0

    lax.fori_loop(0, (NCHUNK - 1) // 2, pair, 0)
    process(NCHUNK - 1, 0)
    for b in range(2):
        basef = pl.multiple_of(wid * EPW + (NCHUNK - 2 + b) * C, 8)
        pltpu.make_async_copy(vbuf[b], vec_hbm.at[pl.ds(basef, C)],
                              sw[b]).wait()


def _sc_vec(pos_flat, src, dst):
    return pl.kernel(
        _sc_vec_body,
        out_type=jax.ShapeDtypeStruct((E, POSW), f32),
        mesh=_mesh,
        compiler_params=_sc_params,
        scratch_types=[
            pltpu.VMEM((N * 4,), f32),
            pltpu.VMEM((C,), jnp.int32),
            pltpu.VMEM((C,), jnp.int32),
            pltpu.VMEM((C,), jnp.int32),
            pltpu.VMEM((C,), jnp.int32),
            pltpu.VMEM((C, POSW), f32),
            pltpu.VMEM((C, POSW), f32),
            pltpu.SemaphoreType.DMA,
            pltpu.SemaphoreType.DMA,
            pltpu.SemaphoreType.DMA,
            pltpu.SemaphoreType.DMA,
            pltpu.SemaphoreType.DMA,
            pltpu.SemaphoreType.DMA,
        ],
    )(pos_flat, src, dst)


def _sc_edge_fwd_body(h_hbm, r_hbm, src_hbm, dst_hbm, agg_hbm,
                      idx_s0, idx_s1, idx_d0, idx_d1, rows0, rows1,
                      rbuf0, rbuf1, agg_sh, sg0, sg1, sr0, sr1, ss0, ss1):
    cid = lax.axis_index("c")
    sid = lax.axis_index("s")
    wid = sid * NC + cid
    idx_s = (idx_s0, idx_s1)
    idx_d = (idx_d0, idx_d1)
    rows = (rows0, rows1)
    rbuf = (rbuf0, rbuf1)
    sg = (sg0, sg1)
    sr = (sr0, sr1)
    ss = (ss0, ss1)
    _zero_shared(agg_sh, rbuf0, sid)
    plsc.subcore_barrier()

    def load(t, b):
        @pl.when(t >= 2)
        def _():
            pltpu.make_async_copy(rbuf[b], agg_sh.at[idx_d[b]], ss[b]).wait()

        base = pl.multiple_of(wid * EPW + t * C, 8)
        pltpu.sync_copy(src_hbm.at[pl.ds(base, C)], idx_s[b])
        pltpu.sync_copy(dst_hbm.at[pl.ds(base, C)], idx_d[b])
        pltpu.async_copy(h_hbm.at[idx_s[b]], rows[b], sg[b])
        pltpu.async_copy(r_hbm.at[pl.ds(base, C)], rbuf[b], sr[b])

    def process(t, b):
        base = pl.multiple_of(wid * EPW + t * C, 8)
        pltpu.make_async_copy(h_hbm.at[idx_s[b]], rows[b], sg[b]).wait()
        pltpu.make_async_copy(r_hbm.at[pl.ds(base, C)], rbuf[b], sr[b]).wait()

        def row(i, _):
            for j in range(H // 16):
                sl = pl.ds(j * 16, 16)
                rbuf[b][i, sl] = rows[b][i, sl] * rbuf[b][i, sl]
            return 0

        lax.fori_loop(0, C, row, 0)
        pltpu.async_copy(rbuf[b], agg_sh.at[idx_d[b]], ss[b], add=True)

    load(0, 0)

    def pair(t2, _):
        for b in range(2):
            t = 2 * t2 + b
            load(t + 1, 1 - b)
            process(t, b)
        return 0

    lax.fori_loop(0, (NCHUNK - 1) // 2, pair, 0)
    process(NCHUNK - 1, 0)
    pltpu.make_async_copy(rbuf[0], agg_sh.at[idx_d[0]], ss[0]).wait()
    pltpu.make_async_copy(rbuf[1], agg_sh.at[idx_d[1]], ss[1]).wait()
    plsc.subcore_barrier()
    _copy_out_shared(agg_sh, agg_hbm, cid, sid)


def _sc_edge_fwd(h, R, src, dst):
    return pl.kernel(
        _sc_edge_fwd_body,
        out_type=jax.ShapeDtypeStruct((NC, N, H), f32),
        mesh=_mesh,
        compiler_params=_sc_params,
        scratch_types=[
            pltpu.VMEM((C,), jnp.int32),
            pltpu.VMEM((C,), jnp.int32),
            pltpu.VMEM((C,), jnp.int32),
            pltpu.VMEM((C,), jnp.int32),
            pltpu.VMEM((C, H), f32),
            pltpu.VMEM((C, H), f32),
            pltpu.VMEM((C, H), f32),
            pltpu.VMEM((C, H), f32),
            pltpu.VMEM_SHARED((N, H), f32),
            pltpu.SemaphoreType.DMA,
            pltpu.SemaphoreType.DMA,
            pltpu.SemaphoreType.DMA,
            pltpu.SemaphoreType.DMA,
            pltpu.SemaphoreType.DMA,
            pltpu.SemaphoreType.DMA,
        ],
    )(h, R, src, dst)


def _sc_edge_bwd1_body(dagg_hbm, h_hbm, r_hbm, src_hbm, dst_hbm, dr_hbm,
                       dh_hbm, idx_s0, idx_s1, idx_d0, idx_d1, idx_st, idx_dt,
                       drows0, drows1, hrows0, hrows1, rbuf0, rbuf1, dh_sh,
                       sa0, sa1, sh0, sh1, sr0, sr1, sw0, sw1, ss0, ss1):
    cid = lax.axis_index("c")
    sid = lax.axis_index("s")
    wid = sid * NC + cid
    idx_s = (idx_s0, idx_s1)
    idx_d = (idx_d0, idx_d1)
    drows = (drows0, drows1)
    hrows = (hrows0, hrows1)
    rbuf = (rbuf0, rbuf1)
    sa = (sa0, sa1)
    sh = (sh0, sh1)
    sr = (sr0, sr1)
    sw = (sw0, sw1)
    ss = (ss0, ss1)
    _zero_shared(dh_sh, rbuf0, sid)
    plsc.subcore_barrier()
    NCH1 = EPW // C1

    def load(t, b):
        @pl.when(t >= 2)
        def _():
            base0 = pl.multiple_of(wid * EPW + (t - 2) * C1, 8)
            pltpu.make_async_copy(hrows[b], dr_hbm.at[pl.ds(base0, C1)],
                                  sw[b]).wait()
            pltpu.make_async_copy(rbuf[b], dh_sh.at[idx_s[b]], ss[b]).wait()

        base = pl.multiple_of(wid * EPW + t * C1, 8)
        pltpu.sync_copy(src_hbm.at[pl.ds(base, C1)], idx_s[b])
        pltpu.sync_copy(dst_hbm.at[pl.ds(base, C1)], idx_d[b])
        pltpu.async_copy(dagg_hbm.at[idx_d[b]], drows[b], sa[b])
        pltpu.async_copy(h_hbm.at[idx_s[b]], hrows[b], sh[b])
        pltpu.async_copy(r_hbm.at[pl.ds(base, C1)], rbuf[b], sr[b])

    def process(t, b):
        base = pl.multiple_of(wid * EPW + t * C1, 8)
        pltpu.make_async_copy(dagg_hbm.at[idx_d[b]], drows[b], sa[b]).wait()
        pltpu.make_async_copy(h_hbm.at[idx_s[b]], hrows[b], sh[b]).wait()
        pltpu.make_async_copy(r_hbm.at[pl.ds(base, C1)], rbuf[b], sr[b]).wait()

        def row(i, _):
            for j in range(H // 16):
                sl = pl.ds(j * 16, 16)
                dm = drows[b][i, sl]
                hrows[b][i, sl] = dm * hrows[b][i, sl]
                rbuf[b][i, sl] = dm * rbuf[b][i, sl]
            return 0

        lax.fori_loop(0, C1, row, 0)
        pltpu.async_copy(hrows[b], dr_hbm.at[pl.ds(base, C1)], sw[b])
        pltpu.async_copy(rbuf[b], dh_sh.at[idx_s[b]], ss[b], add=True)

    load(0, 0)

    def pair(t2, _):
        load(2 * t2 + 1, 1)
        process(2 * t2, 0)

        @pl.when(t2 < (NCH1 // 2) - 1)
        def _():
            load(2 * t2 + 2, 0)

        process(2 * t2 + 1, 1)
        return 0

    lax.fori_loop(0, NCH1 // 2, pair, 0)
    for b in range(2):
        basef = pl.multiple_of(wid * EPW + (NCH1 - 2 + b) * C1, 8)
        pltpu.make_async_copy(hrows[b], dr_hbm.at[pl.ds(basef, C1)],
                              sw[b]).wait()
        pltpu.make_async_copy(rbuf[b], dh_sh.at[idx_s[b]], ss[b]).wait()

    base = pl.multiple_of(wid * EPW + NCH1 * C1, 8)
    pltpu.sync_copy(src_hbm.at[pl.ds(base, CT)], idx_st)
    pltpu.sync_copy(dst_hbm.at[pl.ds(base, CT)], idx_dt)
    pltpu.async_copy(dagg_hbm.at[idx_dt], drows0.at[pl.ds(0, CT)], sa0).wait()
    pltpu.async_copy(h_hbm.at[idx_st], hrows0.at[pl.ds(0, CT)], sh0).wait()
    pltpu.async_copy(r_hbm.at[pl.ds(base, CT)], rbuf0.at[pl.ds(0, CT)],
                     sr0).wait()

    def trow(i, _):
        for j in range(H // 16):
            sl = pl.ds(j * 16, 16)
            dm = drows0[i, sl]
            hrows0[i, sl] = dm * hrows0[i, sl]
            rbuf0[i, sl] = dm * rbuf0[i, sl]
        return 0

    lax.fori_loop(0, CT, trow, 0)
    pltpu.sync_copy(hrows0.at[pl.ds(0, CT)], dr_hbm.at[pl.ds(base, CT)])
    pltpu.sync_copy(rbuf0.at[pl.ds(0, CT)], dh_sh.at[idx_st], add=True)

    plsc.subcore_barrier()
    _copy_out_shared(dh_sh, dh_hbm, cid, sid)


def _sc_edge_bwd1(dagg, h, R, src, dst):
    return pl.kernel(
        _sc_edge_bwd1_body,
        out_type=(jax.ShapeDtypeStruct((E, H), f32),
                  jax.ShapeDtypeStruct((NC, N, H), f32)),
        mesh=_mesh,
        compiler_params=_sc_params,
        scratch_types=[
            pltpu.VMEM((C1,), jnp.int32),
            pltpu.VMEM((C1,), jnp.int32),
            pltpu.VMEM((C1,), jnp.int32),
            pltpu.VMEM((C1,), jnp.int32),
            pltpu.VMEM((CT,), jnp.int32),
            pltpu.VMEM((CT,), jnp.int32),
            pltpu.VMEM((C1, H), f32),
            pltpu.VMEM((C1, H), f32),
            pltpu.VMEM((C1, H), f32),
            pltpu.VMEM((C1, H), f32),
            pltpu.VMEM((C1, H), f32),
            pltpu.VMEM((C1, H), f32),
            pltpu.VMEM_SHARED((N, H), f32),
            pltpu.SemaphoreType.DMA,
            pltpu.SemaphoreType.DMA,
            pltpu.SemaphoreType.DMA,
            pltpu.SemaphoreType.DMA,
            pltpu.SemaphoreType.DMA,
            pltpu.SemaphoreType.DMA,
            pltpu.SemaphoreType.DMA,
            pltpu.SemaphoreType.DMA,
            pltpu.SemaphoreType.DMA,
            pltpu.SemaphoreType.DMA,
        ],
    )(dagg, h, R, src, dst)


def _sc_edge_bwd0_body(dagg_hbm, h_hbm, src_hbm, dst_hbm, dr_hbm,
                       idx_s0, idx_s1, idx_d0, idx_d1, drows0, drows1,
                       hrows0, hrows1, sa0, sa1, sh0, sh1, sw0, sw1):
    wid = _wid()
    idx_s = (idx_s0, idx_s1)
    idx_d = (idx_d0, idx_d1)
    drows = (drows0, drows1)
    hrows = (hrows0, hrows1)
    sa = (sa0, sa1)
    sh = (sh0, sh1)
    sw = (sw0, sw1)

    def load(t, b):
        @pl.when(t >= 2)
        def _():
            base0 = pl.multiple_of(wid * EPW + (t - 2) * C, 8)
            pltpu.make_async_copy(hrows[b], dr_hbm.at[pl.ds(base0, C)],
                                  sw[b]).wait()

        base = pl.multiple_of(wid * EPW + t * C, 8)
        pltpu.sync_copy(src_hbm.at[pl.ds(base, C)], idx_s[b])
        pltpu.sync_copy(dst_hbm.at[pl.ds(base, C)], idx_d[b])
        pltpu.async_copy(dagg_hbm.at[idx_d[b]], drows[b], sa[b])
        pltpu.async_copy(h_hbm.at[idx_s[b]], hrows[b], sh[b])

    def process(t, b):
        base = pl.multiple_of(wid * EPW + t * C, 8)
        pltpu.make_async_copy(dagg_hbm.at[idx_d[b]], drows[b], sa[b]).wait()
        pltpu.make_async_copy(h_hbm.at[idx_s[b]], hrows[b], sh[b]).wait()

        def row(i, _):
            for j in range(H // 16):
                sl = pl.ds(j * 16, 16)
                hrows[b][i, sl] = drows[b][i, sl] * hrows[b][i, sl]
            return 0

        lax.fori_loop(0, C, row, 0)
        pltpu.async_copy(hrows[b], dr_hbm.at[pl.ds(base, C)], sw[b])

    load(0, 0)

    def pair(t2, _):
        for b in range(2):
            t = 2 * t2 + b
            load(t + 1, 1 - b)
            process(t, b)
        return 0

    lax.fori_loop(0, (NCHUNK - 1) // 2, pair, 0)
    process(NCHUNK - 1, 0)
    for b in range(2):
        basef = pl.multiple_of(wid * EPW + (NCHUNK - 2 + b) * C, 8)
        pltpu.make_async_copy(hrows[b], dr_hbm.at[pl.ds(basef, C)],
                              sw[b]).wait()


def _sc_edge_bwd0(dagg, h, src, dst):
    return pl.kernel(
        _sc_edge_bwd0_body,
        out_type=jax.ShapeDtypeStruct((E, H), f32),
        mesh=_mesh,
        compiler_params=_sc_params,
        scratch_types=[
            pltpu.VMEM((C,), jnp.int32),
            pltpu.VMEM((C,), jnp.int32),
            pltpu.VMEM((C,), jnp.int32),
            pltpu.VMEM((C,), jnp.int32),
            pltpu.VMEM((C, H), f32),
            pltpu.VMEM((C, H), f32),
            pltpu.VMEM((C, H), f32),
            pltpu.VMEM((C, H), f32),
            pltpu.SemaphoreType.DMA,
            pltpu.SemaphoreType.DMA,
            pltpu.SemaphoreType.DMA,
            pltpu.SemaphoreType.DMA,
            pltpu.SemaphoreType.DMA,
            pltpu.SemaphoreType.DMA,
        ],
    )(dagg, h, src, dst)


def _sc_forces_body(dv_hbm, src_hbm, dst_hbm, fp_hbm, idx_s, idx_d, dvb,
                    acc, sv):
    wid = _wid()
    zz = jnp.zeros((16,), f32)

    def zrow(k, _):
        acc[pl.ds(k * 16, 16)] = zz
        return 0

    lax.fori_loop(0, (N * 4) // 16, zrow, 0)

    def do_chunk(q):
        base = pl.multiple_of(q * CF, 8)
        pltpu.sync_copy(src_hbm.at[pl.ds(base, CF)], idx_s)
        pltpu.sync_copy(dst_hbm.at[pl.ds(base, CF)], idx_d)
        pltpu.async_copy(dv_hbm.at[:, pl.ds(base, CF)], dvb, sv).wait()

        def group(g, _):
            s16 = idx_s[pl.ds(g * 16, 16)] * 4
            d16 = idx_d[pl.ds(g * 16, 16)] * 4
            for c in range(3):
                vals = dvb[c, pl.ds(g * 16, 16)]
                plsc.addupdate_scatter(acc, [s16 + c], vals)
                plsc.addupdate_scatter(acc, [d16 + c], -vals)
            return 0

        lax.fori_loop(0, CF // 16, group, 0)

    def chunk(t, _):
        do_chunk(wid + NW * t)
        return 0

    lax.fori_loop(0, (E // CF) // NW, chunk, 0)

    @pl.when(wid < (E // CF) % NW)
    def _():
        do_chunk(((E // CF) // NW) * NW + wid)

    pltpu.sync_copy(acc, fp_hbm.at[pl.ds(wid * (N * 4), N * 4)])


def _sc_forces(dvec_t, src, dst):
    return pl.kernel(
        _sc_forces_body,
        out_type=jax.ShapeDtypeStruct((NW * N * 4,), f32),
        mesh=_mesh,
        compiler_params=_sc_params,
        scratch_types=[
            pltpu.VMEM((CF,), jnp.int32),
            pltpu.VMEM((CF,), jnp.int32),
            pltpu.VMEM((4, CF), f32),
            pltpu.VMEM((N * 4,), f32),
            pltpu.SemaphoreType.DMA,
        ],
    )(dvec_t, src, dst)


BN = 1000
BE = 1280


def _silu_prime(z):
    sg = jax.nn.sigmoid(z)
    return sg * (1.0 + z * (1.0 - sg))


def _embed_body(at_ref, w_ref, o_ref):
    t = at_ref[0, 0, :]
    oh = (t[:, None] == lax.broadcasted_iota(jnp.int32, (1, NUM_ELEM), 1)
          ).astype(f32)
    o_ref[...] = jnp.dot(oh, w_ref[...], preferred_element_type=f32)


def _embed(atom_types, W_emb):
    at3 = atom_types.reshape(N // BN, 1, BN)
    return pl.pallas_call(
        _embed_body,
        grid=(N // BN,),
        in_specs=[
            pl.BlockSpec((1, 1, BN), lambda i: (i, 0, 0)),
            pl.BlockSpec((NUM_ELEM, H), lambda i: (0, 0)),
        ],
        out_specs=pl.BlockSpec((BN, H), lambda i: (i, 0)),
        out_shape=jax.ShapeDtypeStruct((N, H), f32),
    )(at3, W_emb)


def _trig1(r):
    rc = jnp.clip(r, 0.0, RMAX)
    t = (math.pi / RMAX) * rc - (math.pi / 2)
    u = t * t
    s1 = 1.0 + u * (-1.0 / 2 + u * (1.0 / 24 + u * (-1.0 / 720 + u * (
        1.0 / 40320 + u * (-1.0 / 3628800 + u * (1.0 / 479001600))))))
    sp = 1.0 + u * (-1.0 / 6 + u * (1.0 / 120 + u * (-1.0 / 5040 + u * (
        1.0 / 362880 + u * (-1.0 / 39916800)))))
    c1 = -t * sp
    return s1, c1


def _harm_sigma(s1, c1, r):
    kn = math.pi / RMAX
    x = kn * jnp.clip(r, 0.0, RMAX)
    sigma1 = kn * jnp.where(x < 0.8, _sincp(x * x), s1 / jnp.maximum(x, 0.5))
    rs1 = r * s1
    sig, cn = sigma1, c1
    sigs, ccs = [sig], [cn]
    for _ in range(NB - 1):
        sig, cn = sig * c1 + cn * sigma1, cn * c1 - sig * rs1
        sigs.append(sig)
        ccs.append(cn)
    return sigs, ccs


def _stack8(cols):
    return jnp.concatenate([x[:, None] for x in cols], axis=1)


def _sincp(w):
    return 1.0 + w * (-1.0 / 6 + w * (1.0 / 120 + w * (-1.0 / 5040 + w * (
        1.0 / 362880))))


def _radial_fwd_body(vec_ref, w1c_ref, w2bd_ref, rbt_ref, vt_ref,
                     r0_ref, r1_ref):
    vt16 = jnp.transpose(vec_ref[...])
    vt = vt16[:4, :]
    s = vt[0] * vt[0] + vt[1] * vt[1] + vt[2] * vt[2] + 1e-12
    r = jnp.sqrt(s)
    s1, c1 = _trig1(r)
    sigs, _ = _harm_sigma(s1, c1, r)
    env = 0.5 * (c1 + 1.0)
    coef = math.sqrt(2.0 / RMAX) * env
    rbt = jnp.concatenate([(coef * sg)[None, :] for sg in sigs], axis=0)
    rbt_ref[...] = rbt
    vt_ref[...] = vt
    aq = lax.dot_general(rbt, w1c_ref[...], (((0,), (0,)), ((), ())),
                         preferred_element_type=f32)
    s01 = aq * jax.nn.sigmoid(aq)
    r01 = jnp.dot(s01, w2bd_ref[...], preferred_element_type=f32)
    r0_ref[...] = r01[:, :H]
    r1_ref[...] = r01[:, H:]


def _radial_fwd(vec16, W1cat, W2bd):
    return pl.pallas_call(
        _radial_fwd_body,
        grid=(E // BE,),
        in_specs=[
            pl.BlockSpec((BE, POSW), lambda i: (i, 0)),
            pl.BlockSpec((NB, 2 * RH), lambda i: (0, 0)),
            pl.BlockSpec((2 * RH, 2 * H), lambda i: (0, 0)),
        ],
        out_specs=[
            pl.BlockSpec((NB, BE), lambda i: (0, i)),
            pl.BlockSpec((4, BE), lambda i: (0, i)),
            pl.BlockSpec((BE, H), lambda i: (i, 0)),
            pl.BlockSpec((BE, H), lambda i: (i, 0)),
        ],
        out_shape=[
            jax.ShapeDtypeStruct((NB, E), f32),
            jax.ShapeDtypeStruct((4, E), f32),
            jax.ShapeDtypeStruct((E, H), f32),
            jax.ShapeDtypeStruct((E, H), f32),
        ],
    )(vec16, W1cat, W2bd)


def _node_fwd_body(aggp_ref, wu_ref, z_ref, h_ref):
    a = aggp_ref[0] + aggp_ref[1]
    z = jnp.dot(a, wu_ref[...], preferred_element_type=f32)
    z_ref[...] = z
    h_ref[...] = z * jax.nn.sigmoid(z)


def _node_fwd(aggp, Wu):
    return pl.pallas_call(
        _node_fwd_body,
        grid=(N // BN,),
        in_specs=[
            pl.BlockSpec((NC, BN, H), lambda i: (0, i, 0)),
            pl.BlockSpec((H, H), lambda i: (0, 0)),
        ],
        out_specs=[
            pl.BlockSpec((BN, H), lambda i: (i, 0)),
            pl.BlockSpec((BN, H), lambda i: (i, 0)),
        ],
        out_shape=[
            jax.ShapeDtypeStruct((N, H), f32),
            jax.ShapeDtypeStruct((N, H), f32),
        ],
    )(aggp, Wu)


def _top_bwd_body(z_ref, h2_ref, wuT_ref, wo_ref, dagg_ref, e_ref):
    i = pl.program_id(0)
    z = z_ref[...]
    wo = wo_ref[...]
    dz = wo * _silu_prime(z)
    dagg_ref[...] = jnp.dot(dz, wuT_ref[...], preferred_element_type=f32)
    part = jnp.sum(h2_ref[...] * wo)

    @pl.when(i == 0)
    def _():
        e_ref[0, 0] = 0.0

    e_ref[0, 0] += part


def _top_bwd(z1, h2, Wu_1T, wo2d):
    return pl.pallas_call(
        _top_bwd_body,
        grid=(N // BN,),
        in_specs=[
            pl.BlockSpec((BN, H), lambda i: (i, 0)),
            pl.BlockSpec((BN, H), lambda i: (i, 0)),
            pl.BlockSpec((H, H), lambda i: (0, 0)),
            pl.BlockSpec((1, H), lambda i: (0, 0)),
        ],
        out_specs=[
            pl.BlockSpec((BN, H), lambda i: (i, 0)),
            pl.BlockSpec(memory_space=pltpu.SMEM),
        ],
        out_shape=[
            jax.ShapeDtypeStruct((N, H), f32),
            jax.ShapeDtypeStruct((1, 1), f32),
        ],
    )(z1, h2, Wu_1T, wo2d)


def _mid_bwd_body(dhp_ref, z_ref, wuT_ref, dagg_ref):
    dh = dhp_ref[0] + dhp_ref[1]
    z = z_ref[...]
    dagg_ref[...] = jnp.dot(dh * _silu_prime(z), wuT_ref[...],
                            preferred_element_type=f32)


def _mid_bwd(dhp, z0, Wu_0T):
    return pl.pallas_call(
        _mid_bwd_body,
        grid=(N // BN,),
        in_specs=[
            pl.BlockSpec((NC, BN, H), lambda i: (0, i, 0)),
            pl.BlockSpec((BN, H), lambda i: (i, 0)),
            pl.BlockSpec((H, H), lambda i: (0, 0)),
        ],
        out_specs=pl.BlockSpec((BN, H), lambda i: (i, 0)),
        out_shape=jax.ShapeDtypeStruct((N, H), f32),
    )(dhp, z0, Wu_0T)


def _radial_bwd_body(dr0_ref, dr1_ref, rbt_ref, vt_ref, w1c_ref, w2bdT_ref,
                     dv_ref):
    vt = vt_ref[...]
    s = vt[0] * vt[0] + vt[1] * vt[1] + vt[2] * vt[2] + 1e-12
    r = jnp.sqrt(s)
    s1, c1 = _trig1(r)
    sigs, ccs = _harm_sigma(s1, c1, r)
    env = 0.5 * (c1 + 1.0)
    rbt = rbt_ref[...]
    aq = lax.dot_general(rbt, w1c_ref[...], (((0,), (0,)), ((), ())),
                         preferred_element_type=f32)
    dr01 = jnp.concatenate([dr0_ref[...], dr1_ref[...]], axis=1)
    ds01 = jnp.dot(dr01, w2bdT_ref[...], preferred_element_type=f32)
    da01 = ds01 * _silu_prime(aq)
    drbt = lax.dot_general(w1c_ref[...], da01, (((1,), (1,)), ((), ())),
                           preferred_element_type=f32)
    c0 = math.sqrt(2.0 / RMAX)
    rinv = 1.0 / r
    kn = math.pi / RMAX
    envp = jnp.where(r < RMAX, -0.5 * kn * s1, 0.0)
    er = env * rinv
    wmat = jnp.concatenate(
        [(c0 * (er * ((n + 1) * kn * cn - sg) + envp * sg))[None, :]
         for n, (sg, cn) in enumerate(zip(sigs, ccs))], axis=0)
    dr = jnp.sum(drbt * wmat, axis=0)
    dv_ref[...] = (dr * rinv)[None, :] * vt


def _radial_bwd(dR0, dR1, rbt, vec_t, W1cat, W2bdT):
    return pl.pallas_call(
        _radial_bwd_body,
        grid=(E // BE,),
        in_specs=[
            pl.BlockSpec((BE, H), lambda i: (i, 0)),
            pl.BlockSpec((BE, H), lambda i: (i, 0)),
            pl.BlockSpec((NB, BE), lambda i: (0, i)),
            pl.BlockSpec((4, BE), lambda i: (0, i)),
            pl.BlockSpec((NB, 2 * RH), lambda i: (0, 0)),
            pl.BlockSpec((2 * H, 2 * RH), lambda i: (0, 0)),
        ],
        out_specs=pl.BlockSpec((4, BE), lambda i: (0, i)),
        out_shape=jax.ShapeDtypeStruct((4, E), f32),
    )(dR0, dR1, rbt, vec_t, W1cat, W2bdT)


FRB = 4096


def _force_reduce_body(fp_ref, o_ref):
    o_ref[...] = jnp.sum(fp_ref[...], axis=0)


def _force_reduce(fp):
    nblk = (N * 4 + FRB - 1) // FRB
    return pl.pallas_call(
        _force_reduce_body,
        grid=(nblk,),
        in_specs=[pl.BlockSpec((NW, FRB), lambda i: (0, i))],
        out_specs=pl.BlockSpec((FRB,), lambda i: (i,)),
        out_shape=jax.ShapeDtypeStruct((N * 4,), f32),
    )(fp)


def kernel(positions, atom_types, edge_index, batch, W_emb, Wr1_0, Wr2_0,
           Wu_0, Wr1_1, Wr2_1, Wu_1, w_out):
    del batch
    pos_flat = jnp.concatenate(
        [positions, jnp.zeros((N, 1), f32)], axis=1).reshape(N * 4)
    src = edge_index[0]
    dst = edge_index[1]

    W1cat = jnp.concatenate([Wr1_0, Wr1_1], axis=1)
    W2bd = jnp.zeros((2 * RH, 2 * H), f32)
    W2bd = W2bd.at[:RH, :H].set(Wr2_0).at[RH:, H:].set(Wr2_1)
    W2bdT = jnp.zeros((2 * H, 2 * RH), f32)
    W2bdT = W2bdT.at[:H, :RH].set(Wr2_0.T).at[H:, RH:].set(Wr2_1.T)

    h0 = _embed(atom_types, W_emb)
    vec16 = _sc_vec(pos_flat, src, dst)
    rbt, vec_t, R0, R1 = _radial_fwd(vec16, W1cat, W2bd)

    aggp0 = _sc_edge_fwd(h0, R0, src, dst)
    z0, h1 = _node_fwd(aggp0, Wu_0)
    aggp1 = _sc_edge_fwd(h1, R1, src, dst)
    z1, h2 = _node_fwd(aggp1, Wu_1)

    dagg1, e11 = _top_bwd(z1, h2, Wu_1.T, w_out.reshape(1, H))
    dR1, dhp1 = _sc_edge_bwd1(dagg1, h1, R1, src, dst)
    dagg0 = _mid_bwd(dhp1, z0, Wu_0.T)
    dR0 = _sc_edge_bwd0(dagg0, h0, src, dst)

    dvec_t = _radial_bwd(dR0, dR1, rbt, vec_t, W1cat, W2bdT)
    fp = _sc_forces(dvec_t, src, dst).reshape(NW, N * 4)
    forces4 = _force_reduce(fp)

    energy = e11.reshape(1)
    forces = forces4.reshape(N, 4)[:, :3]
    return energy, forces


# speedup vs baseline: 1.0643x; 1.0006x over previous
import functools
import math

import jax
import jax.numpy as jnp
from jax import lax
from jax.experimental import pallas as pl
from jax.experimental.pallas import tpu as pltpu
from jax.experimental.pallas import tpu_sc as plsc

N = 10000
E = 320000
NUM_ELEM = 10
H = 128
NB = 8
RH = 64
RMAX = 5.0

NC = 2
NS = 16
NW = NC * NS
EPW = E // NW
C = 80
NCHUNK = EPW // C
NPS = 624
ZR = 48
POSW = 16
C1 = 64
CT = 16
CF = 128

f32 = jnp.float32

_mesh = plsc.VectorSubcoreMesh(
    core_axis_name="c", subcore_axis_name="s", num_cores=NC, num_subcores=NS)
_sc_params = pltpu.CompilerParams(needs_layout_passes=False)


def _wid():
    return lax.axis_index("s") * NC + lax.axis_index("c")


def _zero_shared(sh_ref, zbuf, sid):
    zz = jnp.zeros((16,), f32)

    def zrow(i, _):
        for j in range(H // 16):
            zbuf[i, pl.ds(j * 16, 16)] = zz
        return 0

    lax.fori_loop(0, ZR, zrow, 0)
    for q in range(NPS // ZR):
        pltpu.sync_copy(zbuf.at[pl.ds(0, ZR)],
                        sh_ref.at[pl.ds(sid * NPS + q * ZR, ZR)])

    @pl.when(sid == NS - 1)
    def _():
        pltpu.sync_copy(zbuf.at[pl.ds(0, 16)],
                        sh_ref.at[pl.ds(NS * NPS, 16)])


def _copy_out_shared(sh_ref, out_hbm, cid, sid):
    pltpu.sync_copy(sh_ref.at[pl.ds(sid * NPS, NPS)],
                    out_hbm.at[cid, pl.ds(sid * NPS, NPS)])

    @pl.when(sid == NS - 1)
    def _():
        pltpu.sync_copy(sh_ref.at[pl.ds(NS * NPS, 16)],
                        out_hbm.at[cid, pl.ds(NS * NPS, 16)])


def _sc_vec_body(pos_hbm, src_hbm, dst_hbm, vec_hbm, pos_v, idx_s0, idx_s1,
                 idx_d0, idx_d1, vbuf0, vbuf1, si0, si1, sj0, sj1, sw0, sw1):
    wid = _wid()
    idx_s = (idx_s0, idx_s1)
    idx_d = (idx_d0, idx_d1)
    vbuf = (vbuf0, vbuf1)
    si = (si0, si1)
    sj = (sj0, sj1)
    sw = (sw0, sw1)
    pltpu.sync_copy(pos_hbm, pos_v)
    zz = jnp.zeros((16,), f32)

    def zrow(i, _):
        vbuf0[i, :] = zz
        vbuf1[i, :] = zz
        return 0

    lax.fori_loop(0, C, zrow, 0)

    def load(t, b):
        base = pl.multiple_of(wid * EPW + t * C, 8)
        pltpu.async_copy(src_hbm.at[pl.ds(base, C)], idx_s[b], si[b])
        pltpu.async_copy(dst_hbm.at[pl.ds(base, C)], idx_d[b], sj[b])

    def process(t, b):
        base = pl.multiple_of(wid * EPW + t * C, 8)
        pltpu.make_async_copy(src_hbm.at[pl.ds(base, C)], idx_s[b],
                              si[b]).wait()
        pltpu.make_async_copy(dst_hbm.at[pl.ds(base, C)], idx_d[b],
                              sj[b]).wait()

        @pl.when(t >= 2)
        def _():
            base0 = pl.multiple_of(wid * EPW + (t - 2) * C, 8)
            pltpu.make_async_copy(vbuf[b], vec_hbm.at[pl.ds(base0, C)],
                                  sw[b]).wait()

        def group(g, _):
            rows16 = lax.iota(jnp.int32, 16) + g * 16
            s16 = idx_s[b][pl.ds(g * 16, 16)] * 4
            d16 = idx_d[b][pl.ds(g * 16, 16)] * 4
            for c in range(3):
                ps = plsc.load_gather(pos_v, [s16 + c])
                pd = plsc.load_gather(pos_v, [d16 + c])
                plsc.store_scatter(vbuf[b],
                                   [rows16, jnp.full((16,), c, jnp.int32)],
                                   pd - ps)
            return 0

        lax.fori_loop(0, C // 16, group, 0)
        pltpu.async_copy(vbuf[b], vec_hbm.at[pl.ds(base, C)], sw[b])

    load(0, 0)

    def pair(t2, _):
        for b in range(2):
            t = 2 * t2 + b
            load(t + 1, 1 - b)
            process(t, b)
        return 0

    lax.fori_loop(0, (NCHUNK - 1) // 2, pair, 0)
    process(NCHUNK - 1, 0)
    for b in range(2):
        basef = pl.multiple_of(wid * EPW + (NCHUNK - 2 + b) * C, 8)
        pltpu.make_async_copy(vbuf[b], vec_hbm.at[pl.ds(basef, C)],
                              sw[b]).wait()


def _sc_vec(pos_flat, src, dst):
    return pl.kernel(
        _sc_vec_body,
        out_type=jax.ShapeDtypeStruct((E, POSW), f32),
        mesh=_mesh,
        compiler_params=_sc_params,
        scratch_types=[
            pltpu.VMEM((N * 4,), f32),
            pltpu.VMEM((C,), jnp.int32),
            pltpu.VMEM((C,), jnp.int32),
            pltpu.VMEM((C,), jnp.int32),
            pltpu.VMEM((C,), jnp.int32),
            pltpu.VMEM((C, POSW), f32),
            pltpu.VMEM((C, POSW), f32),
            pltpu.SemaphoreType.DMA,
            pltpu.SemaphoreType.DMA,
            pltpu.SemaphoreType.DMA,
            pltpu.SemaphoreType.DMA,
            pltpu.SemaphoreType.DMA,
            pltpu.SemaphoreType.DMA,
        ],
    )(pos_flat, src, dst)


def _sc_edge_fwd_body(h_hbm, r_hbm, src_hbm, dst_hbm, agg_hbm,
                      idx_s0, idx_s1, idx_d0, idx_d1, rows0, rows1,
                      rbuf0, rbuf1, agg_sh, sg0, sg1, sr0, sr1, ss0, ss1):
    cid = lax.axis_index("c")
    sid = lax.axis_index("s")
    wid = sid * NC + cid
    idx_s = (idx_s0, idx_s1)
    idx_d = (idx_d0, idx_d1)
    rows = (rows0, rows1)
    rbuf = (rbuf0, rbuf1)
    sg = (sg0, sg1)
    sr = (sr0, sr1)
    ss = (ss0, ss1)
    _zero_shared(agg_sh, rbuf0, sid)
    plsc.subcore_barrier()

    def load(t, b):
        @pl.when(t >= 2)
        def _():
            pltpu.make_async_copy(rbuf[b], agg_sh.at[idx_d[b]], ss[b]).wait()

        base = pl.multiple_of(wid * EPW + t * C, 8)
        pltpu.sync_copy(src_hbm.at[pl.ds(base, C)], idx_s[b])
        pltpu.sync_copy(dst_hbm.at[pl.ds(base, C)], idx_d[b])
        pltpu.async_copy(h_hbm.at[idx_s[b]], rows[b], sg[b])
        pltpu.async_copy(r_hbm.at[pl.ds(base, C)], rbuf[b], sr[b])

    def process(t, b):
        base = pl.multiple_of(wid * EPW + t * C, 8)
        pltpu.make_async_copy(h_hbm.at[idx_s[b]], rows[b], sg[b]).wait()
        pltpu.make_async_copy(r_hbm.at[pl.ds(base, C)], rbuf[b], sr[b]).wait()

        def row(i2, _):
            for k in range(2):
                i = i2 * 2 + k
                for j in range(H // 16):
                    sl = pl.ds(j * 16, 16)
                    rbuf[b][i, sl] = rows[b][i, sl] * rbuf[b][i, sl]
            return 0

        lax.fori_loop(0, C // 2, row, 0)
        pltpu.async_copy(rbuf[b], agg_sh.at[idx_d[b]], ss[b], add=True)

    load(0, 0)

    def pair(t2, _):
        for b in range(2):
            t = 2 * t2 + b
            load(t + 1, 1 - b)
            process(t, b)
        return 0

    lax.fori_loop(0, (NCHUNK - 1) // 2, pair, 0)
    process(NCHUNK - 1, 0)
    pltpu.make_async_copy(rbuf[0], agg_sh.at[idx_d[0]], ss[0]).wait()
    pltpu.make_async_copy(rbuf[1], agg_sh.at[idx_d[1]], ss[1]).wait()
    plsc.subcore_barrier()
    _copy_out_shared(agg_sh, agg_hbm, cid, sid)


def _sc_edge_fwd(h, R, src, dst):
    return pl.kernel(
        _sc_edge_fwd_body,
        out_type=jax.ShapeDtypeStruct((NC, N, H), f32),
        mesh=_mesh,
        compiler_params=_sc_params,
        scratch_types=[
            pltpu.VMEM((C,), jnp.int32),
            pltpu.VMEM((C,), jnp.int32),
            pltpu.VMEM((C,), jnp.int32),
            pltpu.VMEM((C,), jnp.int32),
            pltpu.VMEM((C, H), f32),
            pltpu.VMEM((C, H), f32),
            pltpu.VMEM((C, H), f32),
            pltpu.VMEM((C, H), f32),
            pltpu.VMEM_SHARED((N, H), f32),
            pltpu.SemaphoreType.DMA,
            pltpu.SemaphoreType.DMA,
            pltpu.SemaphoreType.DMA,
            pltpu.SemaphoreType.DMA,
            pltpu.SemaphoreType.DMA,
            pltpu.SemaphoreType.DMA,
        ],
    )(h, R, src, dst)


def _sc_edge_bwd1_body(dagg_hbm, h_hbm, r_hbm, src_hbm, dst_hbm, dr_hbm,
                       dh_hbm, idx_s0, idx_s1, idx_d0, idx_d1, idx_st, idx_dt,
                       drows0, drows1, hrows0, hrows1, rbuf0, rbuf1, dh_sh,
                       sa0, sa1, sh0, sh1, sr0, sr1, sw0, sw1, ss0, ss1):
    cid = lax.axis_index("c")
    sid = lax.axis_index("s")
    wid = sid * NC + cid
    idx_s = (idx_s0, idx_s1)
    idx_d = (idx_d0, idx_d1)
    drows = (drows0, drows1)
    hrows = (hrows0, hrows1)
    rbuf = (rbuf0, rbuf1)
    sa = (sa0, sa1)
    sh = (sh0, sh1)
    sr = (sr0, sr1)
    sw = (sw0, sw1)
    ss = (ss0, ss1)
    _zero_shared(dh_sh, rbuf0, sid)
    plsc.subcore_barrier()
    NCH1 = EPW // C1

    def load(t, b):
        @pl.when(t >= 2)
        def _():
            base0 = pl.multiple_of(wid * EPW + (t - 2) * C1, 8)
            pltpu.make_async_copy(hrows[b], dr_hbm.at[pl.ds(base0, C1)],
                                  sw[b]).wait()
            pltpu.make_async_copy(rbuf[b], dh_sh.at[idx_s[b]], ss[b]).wait()

        base = pl.multiple_of(wid * EPW + t * C1, 8)
        pltpu.sync_copy(src_hbm.at[pl.ds(base, C1)], idx_s[b])
        pltpu.sync_copy(dst_hbm.at[pl.ds(base, C1)], idx_d[b])
        pltpu.async_copy(dagg_hbm.at[idx_d[b]], drows[b], sa[b])
        pltpu.async_copy(h_hbm.at[idx_s[b]], hrows[b], sh[b])
        pltpu.async_copy(r_hbm.at[pl.ds(base, C1)], rbuf[b], sr[b])

    def process(t, b):
        base = pl.multiple_of(wid * EPW + t * C1, 8)
        pltpu.make_async_copy(dagg_hbm.at[idx_d[b]], drows[b], sa[b]).wait()
        pltpu.make_async_copy(h_hbm.at[idx_s[b]], hrows[b], sh[b]).wait()
        pltpu.make_async_copy(r_hbm.at[pl.ds(base, C1)], rbuf[b], sr[b]).wait()

        def row(i2, _):
            for k in range(2):
                i = i2 * 2 + k
                for j in range(H // 16):
                    sl = pl.ds(j * 16, 16)
                    dm = drows[b][i, sl]
                    hrows[b][i, sl] = dm * hrows[b][i, sl]
                    rbuf[b][i, sl] = dm * rbuf[b][i, sl]
            return 0

        lax.fori_loop(0, C1 // 2, row, 0)
        pltpu.async_copy(hrows[b], dr_hbm.at[pl.ds(base, C1)], sw[b])
        pltpu.async_copy(rbuf[b], dh_sh.at[idx_s[b]], ss[b], add=True)

    load(0, 0)

    def pair(t2, _):
        load(2 * t2 + 1, 1)
        process(2 * t2, 0)

        @pl.when(t2 < (NCH1 // 2) - 1)
        def _():
            load(2 * t2 + 2, 0)

        process(2 * t2 + 1, 1)
        return 0

    lax.fori_loop(0, NCH1 // 2, pair, 0)
    for b in range(2):
        basef = pl.multiple_of(wid * EPW + (NCH1 - 2 + b) * C1, 8)
        pltpu.make_async_copy(hrows[b], dr_hbm.at[pl.ds(basef, C1)],
                              sw[b]).wait()
        pltpu.make_async_copy(rbuf[b], dh_sh.at[idx_s[b]], ss[b]).wait()

    base = pl.multiple_of(wid * EPW + NCH1 * C1, 8)
    pltpu.sync_copy(src_hbm.at[pl.ds(base, CT)], idx_st)
    pltpu.sync_copy(dst_hbm.at[pl.ds(base, CT)], idx_dt)
    pltpu.async_copy(dagg_hbm.at[idx_dt], drows0.at[pl.ds(0, CT)], sa0).wait()
    pltpu.async_copy(h_hbm.at[idx_st], hrows0.at[pl.ds(0, CT)], sh0).wait()
    pltpu.async_copy(r_hbm.at[pl.ds(base, CT)], rbuf0.at[pl.ds(0, CT)],
                     sr0).wait()

    def trow(i, _):
        for j in range(H // 16):
            sl = pl.ds(j * 16, 16)
            dm = drows0[i, sl]
            hrows0[i, sl] = dm * hrows0[i, sl]
            rbuf0[i, sl] = dm * rbuf0[i, sl]
        return 0

    lax.fori_loop(0, CT, trow, 0)
    pltpu.sync_copy(hrows0.at[pl.ds(0, CT)], dr_hbm.at[pl.ds(base, CT)])
    pltpu.sync_copy(rbuf0.at[pl.ds(0, CT)], dh_sh.at[idx_st], add=True)

    plsc.subcore_barrier()
    _copy_out_shared(dh_sh, dh_hbm, cid, sid)


def _sc_edge_bwd1(dagg, h, R, src, dst):
    return pl.kernel(
        _sc_edge_bwd1_body,
        out_type=(jax.ShapeDtypeStruct((E, H), f32),
                  jax.ShapeDtypeStruct((NC, N, H), f32)),
        mesh=_mesh,
        compiler_params=_sc_params,
        scratch_types=[
            pltpu.VMEM((C1,), jnp.int32),
            pltpu.VMEM((C1,), jnp.int32),
            pltpu.VMEM((C1,), jnp.int32),
            pltpu.VMEM((C1,), jnp.int32),
            pltpu.VMEM((CT,), jnp.int32),
            pltpu.VMEM((CT,), jnp.int32),
            pltpu.VMEM((C1, H), f32),
            pltpu.VMEM((C1, H), f32),
            pltpu.VMEM((C1, H), f32),
            pltpu.VMEM((C1, H), f32),
            pltpu.VMEM((C1, H), f32),
            pltpu.VMEM((C1, H), f32),
            pltpu.VMEM_SHARED((N, H), f32),
            pltpu.SemaphoreType.DMA,
            pltpu.SemaphoreType.DMA,
            pltpu.SemaphoreType.DMA,
            pltpu.SemaphoreType.DMA,
            pltpu.SemaphoreType.DMA,
            pltpu.SemaphoreType.DMA,
            pltpu.SemaphoreType.DMA,
            pltpu.SemaphoreType.DMA,
            pltpu.SemaphoreType.DMA,
            pltpu.SemaphoreType.DMA,
        ],
    )(dagg, h, R, src, dst)


def _sc_edge_bwd0_body(dagg_hbm, h_hbm, src_hbm, dst_hbm, dr_hbm,
                       idx_s0, idx_s1, idx_d0, idx_d1, drows0, drows1,
                       hrows0, hrows1, sa0, sa1, sh0, sh1, sw0, sw1):
    wid = _wid()
    idx_s = (idx_s0, idx_s1)
    idx_d = (idx_d0, idx_d1)
    drows = (drows0, drows1)
    hrows = (hrows0, hrows1)
    sa = (sa0, sa1)
    sh = (sh0, sh1)
    sw = (sw0, sw1)

    def load(t, b):
        @pl.when(t >= 2)
        def _():
            base0 = pl.multiple_of(wid * EPW + (t - 2) * C, 8)
            pltpu.make_async_copy(hrows[b], dr_hbm.at[pl.ds(base0, C)],
                                  sw[b]).wait()

        base = pl.multiple_of(wid * EPW + t * C, 8)
        pltpu.sync_copy(src_hbm.at[pl.ds(base, C)], idx_s[b])
        pltpu.sync_copy(dst_hbm.at[pl.ds(base, C)], idx_d[b])
        pltpu.async_copy(dagg_hbm.at[idx_d[b]], drows[b], sa[b])
        pltpu.async_copy(h_hbm.at[idx_s[b]], hrows[b], sh[b])

    def process(t, b):
        base = pl.multiple_of(wid * EPW + t * C, 8)
        pltpu.make_async_copy(dagg_hbm.at[idx_d[b]], drows[b], sa[b]).wait()
        pltpu.make_async_copy(h_hbm.at[idx_s[b]], hrows[b], sh[b]).wait()

        def row(i2, _):
            for k in range(2):
                i = i2 * 2 + k
                for j in range(H // 16):
                    sl = pl.ds(j * 16, 16)
                    hrows[b][i, sl] = drows[b][i, sl] * hrows[b][i, sl]
            return 0

        lax.fori_loop(0, C // 2, row, 0)
        pltpu.async_copy(hrows[b], dr_hbm.at[pl.ds(base, C)], sw[b])

    load(0, 0)

    def pair(t2, _):
        for b in range(2):
            t = 2 * t2 + b
            load(t + 1, 1 - b)
            process(t, b)
        return 0

    lax.fori_loop(0, (NCHUNK - 1) // 2, pair, 0)
    process(NCHUNK - 1, 0)
    for b in range(2):
        basef = pl.multiple_of(wid * EPW + (NCHUNK - 2 + b) * C, 8)
        pltpu.make_async_copy(hrows[b], dr_hbm.at[pl.ds(basef, C)],
                              sw[b]).wait()


def _sc_edge_bwd0(dagg, h, src, dst):
    return pl.kernel(
        _sc_edge_bwd0_body,
        out_type=jax.ShapeDtypeStruct((E, H), f32),
        mesh=_mesh,
        compiler_params=_sc_params,
        scratch_types=[
            pltpu.VMEM((C,), jnp.int32),
            pltpu.VMEM((C,), jnp.int32),
            pltpu.VMEM((C,), jnp.int32),
            pltpu.VMEM((C,), jnp.int32),
            pltpu.VMEM((C, H), f32),
            pltpu.VMEM((C, H), f32),
            pltpu.VMEM((C, H), f32),
            pltpu.VMEM((C, H), f32),
            pltpu.SemaphoreType.DMA,
            pltpu.SemaphoreType.DMA,
            pltpu.SemaphoreType.DMA,
            pltpu.SemaphoreType.DMA,
            pltpu.SemaphoreType.DMA,
            pltpu.SemaphoreType.DMA,
        ],
    )(dagg, h, src, dst)


def _sc_forces_body(dv_hbm, src_hbm, dst_hbm, fp_hbm, idx_s, idx_d, dvb,
                    acc, sv):
    wid = _wid()
    zz = jnp.zeros((16,), f32)

    def zrow(k, _):
        acc[pl.ds(k * 16, 16)] = zz
        return 0

    lax.fori_loop(0, (N * 4) // 16, zrow, 0)

    def do_chunk(q):
        base = pl.multiple_of(q * CF, 8)
        pltpu.sync_copy(src_hbm.at[pl.ds(base, CF)], idx_s)
        pltpu.sync_copy(dst_hbm.at[pl.ds(base, CF)], idx_d)
        pltpu.async_copy(dv_hbm.at[:, pl.ds(base, CF)], dvb, sv).wait()

        def group(g, _):
            s16 = idx_s[pl.ds(g * 16, 16)] * 4
            d16 = idx_d[pl.ds(g * 16, 16)] * 4
            for c in range(3):
                vals = dvb[c, pl.ds(g * 16, 16)]
                plsc.addupdate_scatter(acc, [s16 + c], vals)
                plsc.addupdate_scatter(acc, [d16 + c], -vals)
            return 0

        lax.fori_loop(0, CF // 16, group, 0)

    def chunk(t, _):
        do_chunk(wid + NW * t)
        return 0

    lax.fori_loop(0, (E // CF) // NW, chunk, 0)

    @pl.when(wid < (E // CF) % NW)
    def _():
        do_chunk(((E // CF) // NW) * NW + wid)

    pltpu.sync_copy(acc, fp_hbm.at[pl.ds(wid * (N * 4), N * 4)])


def _sc_forces(dvec_t, src, dst):
    return pl.kernel(
        _sc_forces_body,
        out_type=jax.ShapeDtypeStruct((NW * N * 4,), f32),
        mesh=_mesh,
        compiler_params=_sc_params,
        scratch_types=[
            pltpu.VMEM((CF,), jnp.int32),
            pltpu.VMEM((CF,), jnp.int32),
            pltpu.VMEM((4, CF), f32),
            pltpu.VMEM((N * 4,), f32),
            pltpu.SemaphoreType.DMA,
        ],
    )(dvec_t, src, dst)


BN = 1000
BE = 1280


def _silu_prime(z):
    sg = jax.nn.sigmoid(z)
    return sg * (1.0 + z * (1.0 - sg))


def _embed_body(at_ref, w_ref, o_ref):
    t = at_ref[0, 0, :]
    oh = (t[:, None] == lax.broadcasted_iota(jnp.int32, (1, NUM_ELEM), 1)
          ).astype(f32)
    o_ref[...] = jnp.dot(oh, w_ref[...], preferred_element_type=f32)


def _embed(atom_types, W_emb):
    at3 = atom_types.reshape(N // BN, 1, BN)
    return pl.pallas_call(
        _embed_body,
        grid=(N // BN,),
        in_specs=[
            pl.BlockSpec((1, 1, BN), lambda i: (i, 0, 0)),
            pl.BlockSpec((NUM_ELEM, H), lambda i: (0, 0)),
        ],
        out_specs=pl.BlockSpec((BN, H), lambda i: (i, 0)),
        out_shape=jax.ShapeDtypeStruct((N, H), f32),
    )(at3, W_emb)


def _trig1(r):
    rc = jnp.clip(r, 0.0, RMAX)
    t = (math.pi / RMAX) * rc - (math.pi / 2)
    u = t * t
    s1 = 1.0 + u * (-1.0 / 2 + u * (1.0 / 24 + u * (-1.0 / 720 + u * (
        1.0 / 40320 + u * (-1.0 / 3628800 + u * (1.0 / 479001600))))))
    sp = 1.0 + u * (-1.0 / 6 + u * (1.0 / 120 + u * (-1.0 / 5040 + u * (
        1.0 / 362880 + u * (-1.0 / 39916800)))))
    c1 = -t * sp
    return s1, c1


def _harm_sigma(s1, c1, r):
    kn = math.pi / RMAX
    x = kn * jnp.clip(r, 0.0, RMAX)
    sigma1 = kn * jnp.where(x < 0.8, _sincp(x * x), s1 / jnp.maximum(x, 0.5))
    rs1 = r * s1
    sig, cn = sigma1, c1
    sigs, ccs = [sig], [cn]
    for _ in range(NB - 1):
        sig, cn = sig * c1 + cn * sigma1, cn * c1 - sig * rs1
        sigs.append(sig)
        ccs.append(cn)
    return sigs, ccs


def _stack8(cols):
    return jnp.concatenate([x[:, None] for x in cols], axis=1)


def _sincp(w):
    return 1.0 + w * (-1.0 / 6 + w * (1.0 / 120 + w * (-1.0 / 5040 + w * (
        1.0 / 362880))))


def _radial_fwd_body(vec_ref, w1c_ref, w2bd_ref, rbt_ref, vt_ref,
                     r0_ref, r1_ref):
    vt16 = jnp.transpose(vec_ref[...])
    vt = vt16[:4, :]
    s = vt[0] * vt[0] + vt[1] * vt[1] + vt[2] * vt[2] + 1e-12
    r = jnp.sqrt(s)
    s1, c1 = _trig1(r)
    sigs, _ = _harm_sigma(s1, c1, r)
    env = 0.5 * (c1 + 1.0)
    coef = math.sqrt(2.0 / RMAX) * env
    rbt = jnp.concatenate([(coef * sg)[None, :] for sg in sigs], axis=0)
    rbt_ref[...] = rbt
    vt_ref[...] = vt
    aq = lax.dot_general(rbt, w1c_ref[...], (((0,), (0,)), ((), ())),
                         preferred_element_type=f32)
    s01 = aq * jax.nn.sigmoid(aq)
    r01 = jnp.dot(s01, w2bd_ref[...], preferred_element_type=f32)
    r0_ref[...] = r01[:, :H]
    r1_ref[...] = r01[:, H:]


def _radial_fwd(vec16, W1cat, W2bd):
    return pl.pallas_call(
        _radial_fwd_body,
        grid=(E // BE,),
        in_specs=[
            pl.BlockSpec((BE, POSW), lambda i: (i, 0)),
            pl.BlockSpec((NB, 2 * RH), lambda i: (0, 0)),
            pl.BlockSpec((2 * RH, 2 * H), lambda i: (0, 0)),
        ],
        out_specs=[
            pl.BlockSpec((NB, BE), lambda i: (0, i)),
            pl.BlockSpec((4, BE), lambda i: (0, i)),
            pl.BlockSpec((BE, H), lambda i: (i, 0)),
            pl.BlockSpec((BE, H), lambda i: (i, 0)),
        ],
        out_shape=[
            jax.ShapeDtypeStruct((NB, E), f32),
            jax.ShapeDtypeStruct((4, E), f32),
            jax.ShapeDtypeStruct((E, H), f32),
            jax.ShapeDtypeStruct((E, H), f32),
        ],
    )(vec16, W1cat, W2bd)


def _node_fwd_body(aggp_ref, wu_ref, z_ref, h_ref):
    a = aggp_ref[0] + aggp_ref[1]
    z = jnp.dot(a, wu_ref[...], preferred_element_type=f32)
    z_ref[...] = z
    h_ref[...] = z * jax.nn.sigmoid(z)


def _node_fwd(aggp, Wu):
    return pl.pallas_call(
        _node_fwd_body,
        grid=(N // BN,),
        in_specs=[
            pl.BlockSpec((NC, BN, H), lambda i: (0, i, 0)),
            pl.BlockSpec((H, H), lambda i: (0, 0)),
        ],
        out_specs=[
            pl.BlockSpec((BN, H), lambda i: (i, 0)),
            pl.BlockSpec((BN, H), lambda i: (i, 0)),
        ],
        out_shape=[
            jax.ShapeDtypeStruct((N, H), f32),
            jax.ShapeDtypeStruct((N, H), f32),
        ],
    )(aggp, Wu)


def _top_bwd_body(z_ref, h2_ref, wuT_ref, wo_ref, dagg_ref, e_ref):
    i = pl.program_id(0)
    z = z_ref[...]
    wo = wo_ref[...]
    dz = wo * _silu_prime(z)
    dagg_ref[...] = jnp.dot(dz, wuT_ref[...], preferred_element_type=f32)
    part = jnp.sum(h2_ref[...] * wo)

    @pl.when(i == 0)
    def _():
        e_ref[0, 0] = 0.0

    e_ref[0, 0] += part


def _top_bwd(z1, h2, Wu_1T, wo2d):
    return pl.pallas_call(
        _top_bwd_body,
        grid=(N // BN,),
        in_specs=[
            pl.BlockSpec((BN, H), lambda i: (i, 0)),
            pl.BlockSpec((BN, H), lambda i: (i, 0)),
            pl.BlockSpec((H, H), lambda i: (0, 0)),
            pl.BlockSpec((1, H), lambda i: (0, 0)),
        ],
        out_specs=[
            pl.BlockSpec((BN, H), lambda i: (i, 0)),
            pl.BlockSpec(memory_space=pltpu.SMEM),
        ],
        out_shape=[
            jax.ShapeDtypeStruct((N, H), f32),
            jax.ShapeDtypeStruct((1, 1), f32),
        ],
    )(z1, h2, Wu_1T, wo2d)


def _mid_bwd_body(dhp_ref, z_ref, wuT_ref, dagg_ref):
    dh = dhp_ref[0] + dhp_ref[1]
    z = z_ref[...]
    dagg_ref[...] = jnp.dot(dh * _silu_prime(z), wuT_ref[...],
                            preferred_element_type=f32)


def _mid_bwd(dhp, z0, Wu_0T):
    return pl.pallas_call(
        _mid_bwd_body,
        grid=(N // BN,),
        in_specs=[
            pl.BlockSpec((NC, BN, H), lambda i: (0, i, 0)),
            pl.BlockSpec((BN, H), lambda i: (i, 0)),
            pl.BlockSpec((H, H), lambda i: (0, 0)),
        ],
        out_specs=pl.BlockSpec((BN, H), lambda i: (i, 0)),
        out_shape=jax.ShapeDtypeStruct((N, H), f32),
    )(dhp, z0, Wu_0T)


def _radial_bwd_body(dr0_ref, dr1_ref, rbt_ref, vt_ref, w1c_ref, w2bdT_ref,
                     dv_ref):
    vt = vt_ref[...]
    s = vt[0] * vt[0] + vt[1] * vt[1] + vt[2] * vt[2] + 1e-12
    r = jnp.sqrt(s)
    s1, c1 = _trig1(r)
    sigs, ccs = _harm_sigma(s1, c1, r)
    env = 0.5 * (c1 + 1.0)
    rbt = rbt_ref[...]
    aq = lax.dot_general(rbt, w1c_ref[...], (((0,), (0,)), ((), ())),
                         preferred_element_type=f32)
    dr01 = jnp.concatenate([dr0_ref[...], dr1_ref[...]], axis=1)
    ds01 = jnp.dot(dr01, w2bdT_ref[...], preferred_element_type=f32)
    da01 = ds01 * _silu_prime(aq)
    drbt = lax.dot_general(w1c_ref[...], da01, (((1,), (1,)), ((), ())),
                           preferred_element_type=f32)
    c0 = math.sqrt(2.0 / RMAX)
    rinv = 1.0 / r
    kn = math.pi / RMAX
    envp = jnp.where(r < RMAX, -0.5 * kn * s1, 0.0)
    er = env * rinv
    wmat = jnp.concatenate(
        [(c0 * (er * ((n + 1) * kn * cn - sg) + envp * sg))[None, :]
         for n, (sg, cn) in enumerate(zip(sigs, ccs))], axis=0)
    dr = jnp.sum(drbt * wmat, axis=0)
    dv_ref[...] = (dr * rinv)[None, :] * vt


def _radial_bwd(dR0, dR1, rbt, vec_t, W1cat, W2bdT):
    return pl.pallas_call(
        _radial_bwd_body,
        grid=(E // BE,),
        in_specs=[
            pl.BlockSpec((BE, H), lambda i: (i, 0)),
            pl.BlockSpec((BE, H), lambda i: (i, 0)),
            pl.BlockSpec((NB, BE), lambda i: (0, i)),
            pl.BlockSpec((4, BE), lambda i: (0, i)),
            pl.BlockSpec((NB, 2 * RH), lambda i: (0, 0)),
            pl.BlockSpec((2 * H, 2 * RH), lambda i: (0, 0)),
        ],
        out_specs=pl.BlockSpec((4, BE), lambda i: (0, i)),
        out_shape=jax.ShapeDtypeStruct((4, E), f32),
    )(dR0, dR1, rbt, vec_t, W1cat, W2bdT)


FRB = 4096


def _force_reduce_body(fp_ref, o_ref):
    o_ref[...] = jnp.sum(fp_ref[...], axis=0)


def _force_reduce(fp):
    nblk = (N * 4 + FRB - 1) // FRB
    return pl.pallas_call(
        _force_reduce_body,
        grid=(nblk,),
        in_specs=[pl.BlockSpec((NW, FRB), lambda i: (0, i))],
        out_specs=pl.BlockSpec((FRB,), lambda i: (i,)),
        out_shape=jax.ShapeDtypeStruct((N * 4,), f32),
    )(fp)


def kernel(positions, atom_types, edge_index, batch, W_emb, Wr1_0, Wr2_0,
           Wu_0, Wr1_1, Wr2_1, Wu_1, w_out):
    del batch
    pos_flat = jnp.concatenate(
        [positions, jnp.zeros((N, 1), f32)], axis=1).reshape(N * 4)
    src = edge_index[0]
    dst = edge_index[1]

    W1cat = jnp.concatenate([Wr1_0, Wr1_1], axis=1)
    W2bd = jnp.zeros((2 * RH, 2 * H), f32)
    W2bd = W2bd.at[:RH, :H].set(Wr2_0).at[RH:, H:].set(Wr2_1)
    W2bdT = jnp.zeros((2 * H, 2 * RH), f32)
    W2bdT = W2bdT.at[:H, :RH].set(Wr2_0.T).at[H:, RH:].set(Wr2_1.T)

    h0 = _embed(atom_types, W_emb)
    vec16 = _sc_vec(pos_flat, src, dst)
    rbt, vec_t, R0, R1 = _radial_fwd(vec16, W1cat, W2bd)

    aggp0 = _sc_edge_fwd(h0, R0, src, dst)
    z0, h1 = _node_fwd(aggp0, Wu_0)
    aggp1 = _sc_edge_fwd(h1, R1, src, dst)
    z1, h2 = _node_fwd(aggp1, Wu_1)

    dagg1, e11 = _top_bwd(z1, h2, Wu_1.T, w_out.reshape(1, H))
    dR1, dhp1 = _sc_edge_bwd1(dagg1, h1, R1, src, dst)
    dagg0 = _mid_bwd(dhp1, z0, Wu_0.T)
    dR0 = _sc_edge_bwd0(dagg0, h0, src, dst)

    dvec_t = _radial_bwd(dR0, dR1, rbt, vec_t, W1cat, W2bdT)
    fp = _sc_forces(dvec_t, src, dst).reshape(NW, N * 4)
    forces4 = _force_reduce(fp)

    energy = e11.reshape(1)
    forces = forces4.reshape(N, 4)[:, :3]
    return energy, forces


# speedup vs baseline: 1.1600x; 1.0900x over previous
import functools
import math

import jax
import jax.numpy as jnp
from jax import lax
from jax.experimental import pallas as pl
from jax.experimental.pallas import tpu as pltpu
from jax.experimental.pallas import tpu_sc as plsc

N = 10000
E = 320000
NUM_ELEM = 10
H = 128
NB = 8
RH = 64
RMAX = 5.0

NC = 2
NS = 16
NW = NC * NS
EPW = E // NW
C = 80
NCHUNK = EPW // C
NPS = 624
ZR = 48
POSW = 16
C1 = 64
CT = 16
CF = 128

f32 = jnp.float32

_mesh = plsc.VectorSubcoreMesh(
    core_axis_name="c", subcore_axis_name="s", num_cores=NC, num_subcores=NS)
_sc_params = pltpu.CompilerParams(needs_layout_passes=False)


def _wid():
    return lax.axis_index("s") * NC + lax.axis_index("c")


def _zero_shared(sh_ref, zbuf, sid):
    zz = jnp.zeros((16,), f32)

    def zrow(i, _):
        for j in range(H // 16):
            zbuf[i, pl.ds(j * 16, 16)] = zz
        return 0

    lax.fori_loop(0, ZR, zrow, 0)
    for q in range(NPS // ZR):
        pltpu.sync_copy(zbuf.at[pl.ds(0, ZR)],
                        sh_ref.at[pl.ds(sid * NPS + q * ZR, ZR)])

    @pl.when(sid == NS - 1)
    def _():
        pltpu.sync_copy(zbuf.at[pl.ds(0, 16)],
                        sh_ref.at[pl.ds(NS * NPS, 16)])


def _copy_out_shared(sh_ref, out_hbm, cid, sid):
    pltpu.sync_copy(sh_ref.at[pl.ds(sid * NPS, NPS)],
                    out_hbm.at[cid, pl.ds(sid * NPS, NPS)])

    @pl.when(sid == NS - 1)
    def _():
        pltpu.sync_copy(sh_ref.at[pl.ds(NS * NPS, 16)],
                        out_hbm.at[cid, pl.ds(NS * NPS, 16)])


def _sc_vec_body(pos_hbm, src_hbm, dst_hbm, vec_hbm, pos_v, idx_s0, idx_s1,
                 idx_d0, idx_d1, vbuf0, vbuf1, si0, si1, sj0, sj1, sw0, sw1):
    wid = _wid()
    idx_s = (idx_s0, idx_s1)
    idx_d = (idx_d0, idx_d1)
    vbuf = (vbuf0, vbuf1)
    si = (si0, si1)
    sj = (sj0, sj1)
    sw = (sw0, sw1)
    pltpu.sync_copy(pos_hbm, pos_v)
    zz = jnp.zeros((16,), f32)

    def zrow(i, _):
        vbuf0[i, :] = zz
        vbuf1[i, :] = zz
        return 0

    lax.fori_loop(0, C, zrow, 0)

    def load(t, b):
        base = pl.multiple_of(wid * EPW + t * C, 8)
        pltpu.async_copy(src_hbm.at[pl.ds(base, C)], idx_s[b], si[b])
        pltpu.async_copy(dst_hbm.at[pl.ds(base, C)], idx_d[b], sj[b])

    def process(t, b):
        base = pl.multiple_of(wid * EPW + t * C, 8)
        pltpu.make_async_copy(src_hbm.at[pl.ds(base, C)], idx_s[b],
                              si[b]).wait()
        pltpu.make_async_copy(dst_hbm.at[pl.ds(base, C)], idx_d[b],
                              sj[b]).wait()

        @pl.when(t >= 2)
        def _():
            base0 = pl.multiple_of(wid * EPW + (t - 2) * C, 8)
            pltpu.make_async_copy(vbuf[b], vec_hbm.at[pl.ds(base0, C)],
                                  sw[b]).wait()

        def group(g, _):
            rows16 = lax.iota(jnp.int32, 16) + g * 16
            s16 = idx_s[b][pl.ds(g * 16, 16)] * 4
            d16 = idx_d[b][pl.ds(g * 16, 16)] * 4
            for c in range(3):
                ps = plsc.load_gather(pos_v, [s16 + c])
                pd = plsc.load_gather(pos_v, [d16 + c])
                plsc.store_scatter(vbuf[b],
                                   [rows16, jnp.full((16,), c, jnp.int32)],
                                   pd - ps)
            return 0

        lax.fori_loop(0, C // 16, group, 0)
        pltpu.async_copy(vbuf[b], vec_hbm.at[pl.ds(base, C)], sw[b])

    load(0, 0)

    def pair(t2, _):
        for b in range(2):
            t = 2 * t2 + b
            load(t + 1, 1 - b)
            process(t, b)
        return 0

    lax.fori_loop(0, (NCHUNK - 1) // 2, pair, 0)
    process(NCHUNK - 1, 0)
    for b in range(2):
        basef = pl.multiple_of(wid * EPW + (NCHUNK - 2 + b) * C, 8)
        pltpu.make_async_copy(vbuf[b], vec_hbm.at[pl.ds(basef, C)],
                              sw[b]).wait()


def _sc_vec(pos_flat, src, dst):
    return pl.kernel(
        _sc_vec_body,
        out_type=jax.ShapeDtypeStruct((E, POSW), f32),
        mesh=_mesh,
        compiler_params=_sc_params,
        scratch_types=[
            pltpu.VMEM((N * 4,), f32),
            pltpu.VMEM((C,), jnp.int32),
            pltpu.VMEM((C,), jnp.int32),
            pltpu.VMEM((C,), jnp.int32),
            pltpu.VMEM((C,), jnp.int32),
            pltpu.VMEM((C, POSW), f32),
            pltpu.VMEM((C, POSW), f32),
            pltpu.SemaphoreType.DMA,
            pltpu.SemaphoreType.DMA,
            pltpu.SemaphoreType.DMA,
            pltpu.SemaphoreType.DMA,
            pltpu.SemaphoreType.DMA,
            pltpu.SemaphoreType.DMA,
        ],
    )(pos_flat, src, dst)


def _sc_edge_fwd_body(h_hbm, r_hbm, src_hbm, dst_hbm, agg_hbm,
                      is0, is1, is2, is3, id0, id1, id2, id3,
                      rows0, rows1, rbuf0, rbuf1, agg_sh,
                      si0, si1, si2, si3, sj0, sj1, sj2, sj3,
                      sg0, sg1, sr0, sr1, ss0, ss1):
    cid = lax.axis_index("c")
    sid = lax.axis_index("s")
    wid = sid * NC + cid
    idx_s = (is0, is1, is2, is3)
    idx_d = (id0, id1, id2, id3)
    rows = (rows0, rows1)
    rbuf = (rbuf0, rbuf1)
    si = (si0, si1, si2, si3)
    sj = (sj0, sj1, sj2, sj3)
    sg = (sg0, sg1)
    sr = (sr0, sr1)
    ss = (ss0, ss1)
    _zero_shared(agg_sh, rbuf0, sid)
    plsc.subcore_barrier()

    def base_of(t):
        return pl.multiple_of(wid * EPW + t * C, 8)

    def idx_load(t, q):
        base = base_of(t)
        pltpu.async_copy(src_hbm.at[pl.ds(base, C)], idx_s[q], si[q])
        pltpu.async_copy(dst_hbm.at[pl.ds(base, C)], idx_d[q], sj[q])

    def idx_wait(t, q):
        base = base_of(t)
        pltpu.make_async_copy(src_hbm.at[pl.ds(base, C)], idx_s[q],
                              si[q]).wait()
        pltpu.make_async_copy(dst_hbm.at[pl.ds(base, C)], idx_d[q],
                              sj[q]).wait()

    def gather_issue(t, b, q):
        pltpu.async_copy(h_hbm.at[idx_s[q]], rows[b], sg[b])
        pltpu.async_copy(r_hbm.at[pl.ds(base_of(t), C)], rbuf[b], sr[b])

    def scatter_drain(b, q):
        pltpu.make_async_copy(rbuf[b], agg_sh.at[idx_d[q]], ss[b]).wait()

    def compute(t, b, q):
        pltpu.make_async_copy(h_hbm.at[idx_s[q]], rows[b], sg[b]).wait()
        pltpu.make_async_copy(r_hbm.at[pl.ds(base_of(t), C)], rbuf[b],
                              sr[b]).wait()

        def row(i2, _):
            for k in range(2):
                i = i2 * 2 + k
                for j in range(H // 16):
                    sl = pl.ds(j * 16, 16)
                    rbuf[b][i, sl] = rows[b][i, sl] * rbuf[b][i, sl]
            return 0

        lax.fori_loop(0, C // 2, row, 0)
        pltpu.async_copy(rbuf[b], agg_sh.at[idx_d[q]], ss[b], add=True)

    idx_load(0, 0)
    idx_load(1, 1)
    idx_wait(0, 0)
    gather_issue(0, 0, 0)

    def quad(t4, _):
        for k in range(4):
            t = 4 * t4 + k
            b = k % 2
            q = k
            qn = (k + 1) % 4
            qn2 = (k + 2) % 4
            if k == 3:
                @pl.when(t4 < (NCHUNK // 4) - 1)
                def _():
                    idx_load(4 * t4 + k + 2, qn2)
            else:
                idx_load(t + 2, qn2)
            idx_wait(t + 1, qn)
            if k == 0:
                @pl.when(t4 > 0)
                def _():
                    scatter_drain(1 - b, (k - 1) % 4)
            else:
                scatter_drain(1 - b, (k - 1) % 4)
            gather_issue(t + 1, 1 - b, qn)
            compute(t, b, q)
        return 0

    lax.fori_loop(0, NCHUNK // 4, quad, 0)
    scatter_drain(1, 3)
    compute(NCHUNK - 1, 0, 0)
    scatter_drain(0, 0)
    plsc.subcore_barrier()
    _copy_out_shared(agg_sh, agg_hbm, cid, sid)


def _sc_edge_fwd(h, R, src, dst):
    return pl.kernel(
        _sc_edge_fwd_body,
        out_type=jax.ShapeDtypeStruct((NC, N, H), f32),
        mesh=_mesh,
        compiler_params=_sc_params,
        scratch_types=(
            [pltpu.VMEM((C,), jnp.int32)] * 8
            + [pltpu.VMEM((C, H), f32)] * 4
            + [pltpu.VMEM_SHARED((N, H), f32)]
            + [pltpu.SemaphoreType.DMA] * 14
        ),
    )(h, R, src, dst)


def _sc_edge_bwd1_body(dagg_hbm, h_hbm, r_hbm, src_hbm, dst_hbm, dr_hbm,
                       dh_hbm, idx_s0, idx_s1, idx_d0, idx_d1, idx_st, idx_dt,
                       drows0, drows1, hrows0, hrows1, rbuf0, rbuf1, dh_sh,
                       sa0, sa1, sh0, sh1, sr0, sr1, sw0, sw1, ss0, ss1):
    cid = lax.axis_index("c")
    sid = lax.axis_index("s")
    wid = sid * NC + cid
    idx_s = (idx_s0, idx_s1)
    idx_d = (idx_d0, idx_d1)
    drows = (drows0, drows1)
    hrows = (hrows0, hrows1)
    rbuf = (rbuf0, rbuf1)
    sa = (sa0, sa1)
    sh = (sh0, sh1)
    sr = (sr0, sr1)
    sw = (sw0, sw1)
    ss = (ss0, ss1)
    _zero_shared(dh_sh, rbuf0, sid)
    plsc.subcore_barrier()
    NCH1 = EPW // C1

    def load(t, b):
        @pl.when(t >= 2)
        def _():
            base0 = pl.multiple_of(wid * EPW + (t - 2) * C1, 8)
            pltpu.make_async_copy(hrows[b], dr_hbm.at[pl.ds(base0, C1)],
                                  sw[b]).wait()
            pltpu.make_async_copy(rbuf[b], dh_sh.at[idx_s[b]], ss[b]).wait()

        base = pl.multiple_of(wid * EPW + t * C1, 8)
        pltpu.sync_copy(src_hbm.at[pl.ds(base, C1)], idx_s[b])
        pltpu.sync_copy(dst_hbm.at[pl.ds(base, C1)], idx_d[b])
        pltpu.async_copy(dagg_hbm.at[idx_d[b]], drows[b], sa[b])
        pltpu.async_copy(h_hbm.at[idx_s[b]], hrows[b], sh[b])
        pltpu.async_copy(r_hbm.at[pl.ds(base, C1)], rbuf[b], sr[b])

    def process(t, b):
        base = pl.multiple_of(wid * EPW + t * C1, 8)
        pltpu.make_async_copy(dagg_hbm.at[idx_d[b]], drows[b], sa[b]).wait()
        pltpu.make_async_copy(h_hbm.at[idx_s[b]], hrows[b], sh[b]).wait()
        pltpu.make_async_copy(r_hbm.at[pl.ds(base, C1)], rbuf[b], sr[b]).wait()

        def row(i2, _):
            for k in range(2):
                i = i2 * 2 + k
                for j in range(H // 16):
                    sl = pl.ds(j * 16, 16)
                    dm = drows[b][i, sl]
                    hrows[b][i, sl] = dm * hrows[b][i, sl]
                    rbuf[b][i, sl] = dm * rbuf[b][i, sl]
            return 0

        lax.fori_loop(0, C1 // 2, row, 0)
        pltpu.async_copy(hrows[b], dr_hbm.at[pl.ds(base, C1)], sw[b])
        pltpu.async_copy(rbuf[b], dh_sh.at[idx_s[b]], ss[b], add=True)

    load(0, 0)

    def pair(t2, _):
        load(2 * t2 + 1, 1)
        process(2 * t2, 0)

        @pl.when(t2 < (NCH1 // 2) - 1)
        def _():
            load(2 * t2 + 2, 0)

        process(2 * t2 + 1, 1)
        return 0

    lax.fori_loop(0, NCH1 // 2, pair, 0)
    for b in range(2):
        basef = pl.multiple_of(wid * EPW + (NCH1 - 2 + b) * C1, 8)
        pltpu.make_async_copy(hrows[b], dr_hbm.at[pl.ds(basef, C1)],
                              sw[b]).wait()
        pltpu.make_async_copy(rbuf[b], dh_sh.at[idx_s[b]], ss[b]).wait()

    base = pl.multiple_of(wid * EPW + NCH1 * C1, 8)
    pltpu.sync_copy(src_hbm.at[pl.ds(base, CT)], idx_st)
    pltpu.sync_copy(dst_hbm.at[pl.ds(base, CT)], idx_dt)
    pltpu.async_copy(dagg_hbm.at[idx_dt], drows0.at[pl.ds(0, CT)], sa0).wait()
    pltpu.async_copy(h_hbm.at[idx_st], hrows0.at[pl.ds(0, CT)], sh0).wait()
    pltpu.async_copy(r_hbm.at[pl.ds(base, CT)], rbuf0.at[pl.ds(0, CT)],
                     sr0).wait()

    def trow(i, _):
        for j in range(H // 16):
            sl = pl.ds(j * 16, 16)
            dm = drows0[i, sl]
            hrows0[i, sl] = dm * hrows0[i, sl]
            rbuf0[i, sl] = dm * rbuf0[i, sl]
        return 0

    lax.fori_loop(0, CT, trow, 0)
    pltpu.sync_copy(hrows0.at[pl.ds(0, CT)], dr_hbm.at[pl.ds(base, CT)])
    pltpu.sync_copy(rbuf0.at[pl.ds(0, CT)], dh_sh.at[idx_st], add=True)

    plsc.subcore_barrier()
    _copy_out_shared(dh_sh, dh_hbm, cid, sid)


def _sc_edge_bwd1(dagg, h, R, src, dst):
    return pl.kernel(
        _sc_edge_bwd1_body,
        out_type=(jax.ShapeDtypeStruct((E, H), f32),
                  jax.ShapeDtypeStruct((NC, N, H), f32)),
        mesh=_mesh,
        compiler_params=_sc_params,
        scratch_types=[
            pltpu.VMEM((C1,), jnp.int32),
            pltpu.VMEM((C1,), jnp.int32),
            pltpu.VMEM((C1,), jnp.int32),
            pltpu.VMEM((C1,), jnp.int32),
            pltpu.VMEM((CT,), jnp.int32),
            pltpu.VMEM((CT,), jnp.int32),
            pltpu.VMEM((C1, H), f32),
            pltpu.VMEM((C1, H), f32),
            pltpu.VMEM((C1, H), f32),
            pltpu.VMEM((C1, H), f32),
            pltpu.VMEM((C1, H), f32),
            pltpu.VMEM((C1, H), f32),
            pltpu.VMEM_SHARED((N, H), f32),
            pltpu.SemaphoreType.DMA,
            pltpu.SemaphoreType.DMA,
            pltpu.SemaphoreType.DMA,
            pltpu.SemaphoreType.DMA,
            pltpu.SemaphoreType.DMA,
            pltpu.SemaphoreType.DMA,
            pltpu.SemaphoreType.DMA,
            pltpu.SemaphoreType.DMA,
            pltpu.SemaphoreType.DMA,
            pltpu.SemaphoreType.DMA,
        ],
    )(dagg, h, R, src, dst)


def _sc_edge_bwd0_body(dagg_hbm, h_hbm, src_hbm, dst_hbm, dr_hbm,
                       idx_s0, idx_s1, idx_d0, idx_d1, drows0, drows1,
                       hrows0, hrows1, sa0, sa1, sh0, sh1, sw0, sw1):
    wid = _wid()
    idx_s = (idx_s0, idx_s1)
    idx_d = (idx_d0, idx_d1)
    drows = (drows0, drows1)
    hrows = (hrows0, hrows1)
    sa = (sa0, sa1)
    sh = (sh0, sh1)
    sw = (sw0, sw1)

    def load(t, b):
        @pl.when(t >= 2)
        def _():
            base0 = pl.multiple_of(wid * EPW + (t - 2) * C, 8)
            pltpu.make_async_copy(hrows[b], dr_hbm.at[pl.ds(base0, C)],
                                  sw[b]).wait()

        base = pl.multiple_of(wid * EPW + t * C, 8)
        pltpu.sync_copy(src_hbm.at[pl.ds(base, C)], idx_s[b])
        pltpu.sync_copy(dst_hbm.at[pl.ds(base, C)], idx_d[b])
        pltpu.async_copy(dagg_hbm.at[idx_d[b]], drows[b], sa[b])
        pltpu.async_copy(h_hbm.at[idx_s[b]], hrows[b], sh[b])

    def process(t, b):
        base = pl.multiple_of(wid * EPW + t * C, 8)
        pltpu.make_async_copy(dagg_hbm.at[idx_d[b]], drows[b], sa[b]).wait()
        pltpu.make_async_copy(h_hbm.at[idx_s[b]], hrows[b], sh[b]).wait()

        def row(i2, _):
            for k in range(2):
                i = i2 * 2 + k
                for j in range(H // 16):
                    sl = pl.ds(j * 16, 16)
                    hrows[b][i, sl] = drows[b][i, sl] * hrows[b][i, sl]
            return 0

        lax.fori_loop(0, C // 2, row, 0)
        pltpu.async_copy(hrows[b], dr_hbm.at[pl.ds(base, C)], sw[b])

    load(0, 0)

    def pair(t2, _):
        for b in range(2):
            t = 2 * t2 + b
            load(t + 1, 1 - b)
            process(t, b)
        return 0

    lax.fori_loop(0, (NCHUNK - 1) // 2, pair, 0)
    process(NCHUNK - 1, 0)
    for b in range(2):
        basef = pl.multiple_of(wid * EPW + (NCHUNK - 2 + b) * C, 8)
        pltpu.make_async_copy(hrows[b], dr_hbm.at[pl.ds(basef, C)],
                              sw[b]).wait()


def _sc_edge_bwd0(dagg, h, src, dst):
    return pl.kernel(
        _sc_edge_bwd0_body,
        out_type=jax.ShapeDtypeStruct((E, H), f32),
        mesh=_mesh,
        compiler_params=_sc_params,
        scratch_types=[
            pltpu.VMEM((C,), jnp.int32),
            pltpu.VMEM((C,), jnp.int32),
            pltpu.VMEM((C,), jnp.int32),
            pltpu.VMEM((C,), jnp.int32),
            pltpu.VMEM((C, H), f32),
            pltpu.VMEM((C, H), f32),
            pltpu.VMEM((C, H), f32),
            pltpu.VMEM((C, H), f32),
            pltpu.SemaphoreType.DMA,
            pltpu.SemaphoreType.DMA,
            pltpu.SemaphoreType.DMA,
            pltpu.SemaphoreType.DMA,
            pltpu.SemaphoreType.DMA,
            pltpu.SemaphoreType.DMA,
        ],
    )(dagg, h, src, dst)


def _sc_forces_body(dv_hbm, src_hbm, dst_hbm, fp_hbm, idx_s, idx_d, dvb,
                    acc, sv):
    wid = _wid()
    zz = jnp.zeros((16,), f32)

    def zrow(k, _):
        acc[pl.ds(k * 16, 16)] = zz
        return 0

    lax.fori_loop(0, (N * 4) // 16, zrow, 0)

    def do_chunk(q):
        base = pl.multiple_of(q * CF, 8)
        pltpu.sync_copy(src_hbm.at[pl.ds(base, CF)], idx_s)
        pltpu.sync_copy(dst_hbm.at[pl.ds(base, CF)], idx_d)
        pltpu.async_copy(dv_hbm.at[:, pl.ds(base, CF)], dvb, sv).wait()

        def group(g, _):
            s16 = idx_s[pl.ds(g * 16, 16)] * 4
            d16 = idx_d[pl.ds(g * 16, 16)] * 4
            for c in range(3):
                vals = dvb[c, pl.ds(g * 16, 16)]
                plsc.addupdate_scatter(acc, [s16 + c], vals)
                plsc.addupdate_scatter(acc, [d16 + c], -vals)
            return 0

        lax.fori_loop(0, CF // 16, group, 0)

    def chunk(t, _):
        do_chunk(wid + NW * t)
        return 0

    lax.fori_loop(0, (E // CF) // NW, chunk, 0)

    @pl.when(wid < (E // CF) % NW)
    def _():
        do_chunk(((E // CF) // NW) * NW + wid)

    pltpu.sync_copy(acc, fp_hbm.at[pl.ds(wid * (N * 4), N * 4)])


def _sc_forces(dvec_t, src, dst):
    return pl.kernel(
        _sc_forces_body,
        out_type=jax.ShapeDtypeStruct((NW * N * 4,), f32),
        mesh=_mesh,
        compiler_params=_sc_params,
        scratch_types=[
            pltpu.VMEM((CF,), jnp.int32),
            pltpu.VMEM((CF,), jnp.int32),
            pltpu.VMEM((4, CF), f32),
            pltpu.VMEM((N * 4,), f32),
            pltpu.SemaphoreType.DMA,
        ],
    )(dvec_t, src, dst)


BN = 1000
BE = 1280


def _silu_prime(z):
    sg = jax.nn.sigmoid(z)
    return sg * (1.0 + z * (1.0 - sg))


def _embed_body(at_ref, w_ref, o_ref):
    t = at_ref[0, 0, :]
    oh = (t[:, None] == lax.broadcasted_iota(jnp.int32, (1, NUM_ELEM), 1)
          ).astype(f32)
    o_ref[...] = jnp.dot(oh, w_ref[...], preferred_element_type=f32)


def _embed(atom_types, W_emb):
    at3 = atom_types.reshape(N // BN, 1, BN)
    return pl.pallas_call(
        _embed_body,
        grid=(N // BN,),
        in_specs=[
            pl.BlockSpec((1, 1, BN), lambda i: (i, 0, 0)),
            pl.BlockSpec((NUM_ELEM, H), lambda i: (0, 0)),
        ],
        out_specs=pl.BlockSpec((BN, H), lambda i: (i, 0)),
        out_shape=jax.ShapeDtypeStruct((N, H), f32),
    )(at3, W_emb)


def _trig1(r):
    rc = jnp.clip(r, 0.0, RMAX)
    t = (math.pi / RMAX) * rc - (math.pi / 2)
    u = t * t
    s1 = 1.0 + u * (-1.0 / 2 + u * (1.0 / 24 + u * (-1.0 / 720 + u * (
        1.0 / 40320 + u * (-1.0 / 3628800 + u * (1.0 / 479001600))))))
    sp = 1.0 + u * (-1.0 / 6 + u * (1.0 / 120 + u * (-1.0 / 5040 + u * (
        1.0 / 362880 + u * (-1.0 / 39916800)))))
    c1 = -t * sp
    return s1, c1


def _harm_sigma(s1, c1, r):
    kn = math.pi / RMAX
    x = kn * jnp.clip(r, 0.0, RMAX)
    sigma1 = kn * jnp.where(x < 0.8, _sincp(x * x), s1 / jnp.maximum(x, 0.5))
    rs1 = r * s1
    sig, cn = sigma1, c1
    sigs, ccs = [sig], [cn]
    for _ in range(NB - 1):
        sig, cn = sig * c1 + cn * sigma1, cn * c1 - sig * rs1
        sigs.append(sig)
        ccs.append(cn)
    return sigs, ccs


def _stack8(cols):
    return jnp.concatenate([x[:, None] for x in cols], axis=1)


def _sincp(w):
    return 1.0 + w * (-1.0 / 6 + w * (1.0 / 120 + w * (-1.0 / 5040 + w * (
        1.0 / 362880))))


def _radial_fwd_body(vec_ref, w1c_ref, w2bd_ref, rbt_ref, vt_ref,
                     r0_ref, r1_ref):
    vt16 = jnp.transpose(vec_ref[...])
    vt = vt16[:4, :]
    s = vt[0] * vt[0] + vt[1] * vt[1] + vt[2] * vt[2] + 1e-12
    r = jnp.sqrt(s)
    s1, c1 = _trig1(r)
    sigs, _ = _harm_sigma(s1, c1, r)
    env = 0.5 * (c1 + 1.0)
    coef = math.sqrt(2.0 / RMAX) * env
    rbt = jnp.concatenate([(coef * sg)[None, :] for sg in sigs], axis=0)
    rbt_ref[...] = rbt
    vt_ref[...] = vt
    aq = lax.dot_general(rbt, w1c_ref[...], (((0,), (0,)), ((), ())),
                         preferred_element_type=f32)
    s01 = aq * jax.nn.sigmoid(aq)
    r01 = jnp.dot(s01, w2bd_ref[...], preferred_element_type=f32)
    r0_ref[...] = r01[:, :H]
    r1_ref[...] = r01[:, H:]


def _radial_fwd(vec16, W1cat, W2bd):
    return pl.pallas_call(
        _radial_fwd_body,
        grid=(E // BE,),
        in_specs=[
            pl.BlockSpec((BE, POSW), lambda i: (i, 0)),
            pl.BlockSpec((NB, 2 * RH), lambda i: (0, 0)),
            pl.BlockSpec((2 * RH, 2 * H), lambda i: (0, 0)),
        ],
        out_specs=[
            pl.BlockSpec((NB, BE), lambda i: (0, i)),
            pl.BlockSpec((4, BE), lambda i: (0, i)),
            pl.BlockSpec((BE, H), lambda i: (i, 0)),
            pl.BlockSpec((BE, H), lambda i: (i, 0)),
        ],
        out_shape=[
            jax.ShapeDtypeStruct((NB, E), f32),
            jax.ShapeDtypeStruct((4, E), f32),
            jax.ShapeDtypeStruct((E, H), f32),
            jax.ShapeDtypeStruct((E, H), f32),
        ],
    )(vec16, W1cat, W2bd)


def _node_fwd_body(aggp_ref, wu_ref, z_ref, h_ref):
    a = aggp_ref[0] + aggp_ref[1]
    z = jnp.dot(a, wu_ref[...], preferred_element_type=f32)
    z_ref[...] = z
    h_ref[...] = z * jax.nn.sigmoid(z)


def _node_fwd(aggp, Wu):
    return pl.pallas_call(
        _node_fwd_body,
        grid=(N // BN,),
        in_specs=[
            pl.BlockSpec((NC, BN, H), lambda i: (0, i, 0)),
            pl.BlockSpec((H, H), lambda i: (0, 0)),
        ],
        out_specs=[
            pl.BlockSpec((BN, H), lambda i: (i, 0)),
            pl.BlockSpec((BN, H), lambda i: (i, 0)),
        ],
        out_shape=[
            jax.ShapeDtypeStruct((N, H), f32),
            jax.ShapeDtypeStruct((N, H), f32),
        ],
    )(aggp, Wu)


def _top_bwd_body(z_ref, h2_ref, wuT_ref, wo_ref, dagg_ref, e_ref):
    i = pl.program_id(0)
    z = z_ref[...]
    wo = wo_ref[...]
    dz = wo * _silu_prime(z)
    dagg_ref[...] = jnp.dot(dz, wuT_ref[...], preferred_element_type=f32)
    part = jnp.sum(h2_ref[...] * wo)

    @pl.when(i == 0)
    def _():
        e_ref[0, 0] = 0.0

    e_ref[0, 0] += part


def _top_bwd(z1, h2, Wu_1T, wo2d):
    return pl.pallas_call(
        _top_bwd_body,
        grid=(N // BN,),
        in_specs=[
            pl.BlockSpec((BN, H), lambda i: (i, 0)),
            pl.BlockSpec((BN, H), lambda i: (i, 0)),
            pl.BlockSpec((H, H), lambda i: (0, 0)),
            pl.BlockSpec((1, H), lambda i: (0, 0)),
        ],
        out_specs=[
            pl.BlockSpec((BN, H), lambda i: (i, 0)),
            pl.BlockSpec(memory_space=pltpu.SMEM),
        ],
        out_shape=[
            jax.ShapeDtypeStruct((N, H), f32),
            jax.ShapeDtypeStruct((1, 1), f32),
        ],
    )(z1, h2, Wu_1T, wo2d)


def _mid_bwd_body(dhp_ref, z_ref, wuT_ref, dagg_ref):
    dh = dhp_ref[0] + dhp_ref[1]
    z = z_ref[...]
    dagg_ref[...] = jnp.dot(dh * _silu_prime(z), wuT_ref[...],
                            preferred_element_type=f32)


def _mid_bwd(dhp, z0, Wu_0T):
    return pl.pallas_call(
        _mid_bwd_body,
        grid=(N // BN,),
        in_specs=[
            pl.BlockSpec((NC, BN, H), lambda i: (0, i, 0)),
            pl.BlockSpec((BN, H), lambda i: (i, 0)),
            pl.BlockSpec((H, H), lambda i: (0, 0)),
        ],
        out_specs=pl.BlockSpec((BN, H), lambda i: (i, 0)),
        out_shape=jax.ShapeDtypeStruct((N, H), f32),
    )(dhp, z0, Wu_0T)


def _radial_bwd_body(dr0_ref, dr1_ref, rbt_ref, vt_ref, w1c_ref, w2bdT_ref,
                     dv_ref):
    vt = vt_ref[...]
    s = vt[0] * vt[0] + vt[1] * vt[1] + vt[2] * vt[2] + 1e-12
    r = jnp.sqrt(s)
    s1, c1 = _trig1(r)
    sigs, ccs = _harm_sigma(s1, c1, r)
    env = 0.5 * (c1 + 1.0)
    rbt = rbt_ref[...]
    aq = lax.dot_general(rbt, w1c_ref[...], (((0,), (0,)), ((), ())),
                         preferred_element_type=f32)
    dr01 = jnp.concatenate([dr0_ref[...], dr1_ref[...]], axis=1)
    ds01 = jnp.dot(dr01, w2bdT_ref[...], preferred_element_type=f32)
    da01 = ds01 * _silu_prime(aq)
    drbt = lax.dot_general(w1c_ref[...], da01, (((1,), (1,)), ((), ())),
                           preferred_element_type=f32)
    c0 = math.sqrt(2.0 / RMAX)
    rinv = 1.0 / r
    kn = math.pi / RMAX
    envp = jnp.where(r < RMAX, -0.5 * kn * s1, 0.0)
    er = env * rinv
    wmat = jnp.concatenate(
        [(c0 * (er * ((n + 1) * kn * cn - sg) + envp * sg))[None, :]
         for n, (sg, cn) in enumerate(zip(sigs, ccs))], axis=0)
    dr = jnp.sum(drbt * wmat, axis=0)
    dv_ref[...] = (dr * rinv)[None, :] * vt


def _radial_bwd(dR0, dR1, rbt, vec_t, W1cat, W2bdT):
    return pl.pallas_call(
        _radial_bwd_body,
        grid=(E // BE,),
        in_specs=[
            pl.BlockSpec((BE, H), lambda i: (i, 0)),
            pl.BlockSpec((BE, H), lambda i: (i, 0)),
            pl.BlockSpec((NB, BE), lambda i: (0, i)),
            pl.BlockSpec((4, BE), lambda i: (0, i)),
            pl.BlockSpec((NB, 2 * RH), lambda i: (0, 0)),
            pl.BlockSpec((2 * H, 2 * RH), lambda i: (0, 0)),
        ],
        out_specs=pl.BlockSpec((4, BE), lambda i: (0, i)),
        out_shape=jax.ShapeDtypeStruct((4, E), f32),
    )(dR0, dR1, rbt, vec_t, W1cat, W2bdT)


FRB = 4096


def _force_reduce_body(fp_ref, o_ref):
    o_ref[...] = jnp.sum(fp_ref[...], axis=0)


def _force_reduce(fp):
    nblk = (N * 4 + FRB - 1) // FRB
    return pl.pallas_call(
        _force_reduce_body,
        grid=(nblk,),
        in_specs=[pl.BlockSpec((NW, FRB), lambda i: (0, i))],
        out_specs=pl.BlockSpec((FRB,), lambda i: (i,)),
        out_shape=jax.ShapeDtypeStruct((N * 4,), f32),
    )(fp)


def kernel(positions, atom_types, edge_index, batch, W_emb, Wr1_0, Wr2_0,
           Wu_0, Wr1_1, Wr2_1, Wu_1, w_out):
    del batch
    pos_flat = jnp.concatenate(
        [positions, jnp.zeros((N, 1), f32)], axis=1).reshape(N * 4)
    src = edge_index[0]
    dst = edge_index[1]

    W1cat = jnp.concatenate([Wr1_0, Wr1_1], axis=1)
    W2bd = jnp.zeros((2 * RH, 2 * H), f32)
    W2bd = W2bd.at[:RH, :H].set(Wr2_0).at[RH:, H:].set(Wr2_1)
    W2bdT = jnp.zeros((2 * H, 2 * RH), f32)
    W2bdT = W2bdT.at[:H, :RH].set(Wr2_0.T).at[H:, RH:].set(Wr2_1.T)

    h0 = _embed(atom_types, W_emb)
    vec16 = _sc_vec(pos_flat, src, dst)
    rbt, vec_t, R0, R1 = _radial_fwd(vec16, W1cat, W2bd)

    aggp0 = _sc_edge_fwd(h0, R0, src, dst)
    z0, h1 = _node_fwd(aggp0, Wu_0)
    aggp1 = _sc_edge_fwd(h1, R1, src, dst)
    z1, h2 = _node_fwd(aggp1, Wu_1)

    dagg1, e11 = _top_bwd(z1, h2, Wu_1.T, w_out.reshape(1, H))
    dR1, dhp1 = _sc_edge_bwd1(dagg1, h1, R1, src, dst)
    dagg0 = _mid_bwd(dhp1, z0, Wu_0.T)
    dR0 = _sc_edge_bwd0(dagg0, h0, src, dst)

    dvec_t = _radial_bwd(dR0, dR1, rbt, vec_t, W1cat, W2bdT)
    fp = _sc_forces(dvec_t, src, dst).reshape(NW, N * 4)
    forces4 = _force_reduce(fp)

    energy = e11.reshape(1)
    forces = forces4.reshape(N, 4)[:, :3]
    return energy, forces


# speedup vs baseline: 1.2090x; 1.0422x over previous
import functools
import math

import jax
import jax.numpy as jnp
from jax import lax
from jax.experimental import pallas as pl
from jax.experimental.pallas import tpu as pltpu
from jax.experimental.pallas import tpu_sc as plsc

N = 10000
E = 320000
NUM_ELEM = 10
H = 128
NB = 8
RH = 64
RMAX = 5.0

NC = 2
NS = 16
NW = NC * NS
EPW = E // NW
C = 80
NCHUNK = EPW // C
NPS = 624
ZR = 48
POSW = 16
C1 = 64
CT = 16
CF = 128

f32 = jnp.float32

_mesh = plsc.VectorSubcoreMesh(
    core_axis_name="c", subcore_axis_name="s", num_cores=NC, num_subcores=NS)
_sc_params = pltpu.CompilerParams(needs_layout_passes=False)


def _wid():
    return lax.axis_index("s") * NC + lax.axis_index("c")


def _zero_shared(sh_ref, zbuf, sid):
    zz = jnp.zeros((16,), f32)

    def zrow(i, _):
        for j in range(H // 16):
            zbuf[i, pl.ds(j * 16, 16)] = zz
        return 0

    lax.fori_loop(0, ZR, zrow, 0)
    for q in range(NPS // ZR):
        pltpu.sync_copy(zbuf.at[pl.ds(0, ZR)],
                        sh_ref.at[pl.ds(sid * NPS + q * ZR, ZR)])

    @pl.when(sid == NS - 1)
    def _():
        pltpu.sync_copy(zbuf.at[pl.ds(0, 16)],
                        sh_ref.at[pl.ds(NS * NPS, 16)])


def _copy_out_shared(sh_ref, out_hbm, cid, sid):
    pltpu.sync_copy(sh_ref.at[pl.ds(sid * NPS, NPS)],
                    out_hbm.at[cid, pl.ds(sid * NPS, NPS)])

    @pl.when(sid == NS - 1)
    def _():
        pltpu.sync_copy(sh_ref.at[pl.ds(NS * NPS, 16)],
                        out_hbm.at[cid, pl.ds(NS * NPS, 16)])


def _sc_vec_body(pos_hbm, src_hbm, dst_hbm, vec_hbm, pos_v, idx_s0, idx_s1,
                 idx_d0, idx_d1, vbuf0, vbuf1, si0, si1, sj0, sj1, sw0, sw1):
    wid = _wid()
    idx_s = (idx_s0, idx_s1)
    idx_d = (idx_d0, idx_d1)
    vbuf = (vbuf0, vbuf1)
    si = (si0, si1)
    sj = (sj0, sj1)
    sw = (sw0, sw1)
    pltpu.sync_copy(pos_hbm, pos_v)
    zz = jnp.zeros((16,), f32)

    def zrow(i, _):
        vbuf0[i, :] = zz
        vbuf1[i, :] = zz
        return 0

    lax.fori_loop(0, C, zrow, 0)

    def load(t, b):
        base = pl.multiple_of(wid * EPW + t * C, 8)
        pltpu.async_copy(src_hbm.at[pl.ds(base, C)], idx_s[b], si[b])
        pltpu.async_copy(dst_hbm.at[pl.ds(base, C)], idx_d[b], sj[b])

    def process(t, b):
        base = pl.multiple_of(wid * EPW + t * C, 8)
        pltpu.make_async_copy(src_hbm.at[pl.ds(base, C)], idx_s[b],
                              si[b]).wait()
        pltpu.make_async_copy(dst_hbm.at[pl.ds(base, C)], idx_d[b],
                              sj[b]).wait()

        @pl.when(t >= 2)
        def _():
            base0 = pl.multiple_of(wid * EPW + (t - 2) * C, 8)
            pltpu.make_async_copy(vbuf[b], vec_hbm.at[pl.ds(base0, C)],
                                  sw[b]).wait()

        def group(g, _):
            rows16 = lax.iota(jnp.int32, 16) + g * 16
            s16 = idx_s[b][pl.ds(g * 16, 16)] * 4
            d16 = idx_d[b][pl.ds(g * 16, 16)] * 4
            for c in range(3):
                ps = plsc.load_gather(pos_v, [s16 + c])
                pd = plsc.load_gather(pos_v, [d16 + c])
                plsc.store_scatter(vbuf[b],
                                   [rows16, jnp.full((16,), c, jnp.int32)],
                                   pd - ps)
            return 0

        lax.fori_loop(0, C // 16, group, 0)
        pltpu.async_copy(vbuf[b], vec_hbm.at[pl.ds(base, C)], sw[b])

    load(0, 0)

    def pair(t2, _):
        for b in range(2):
            t = 2 * t2 + b
            load(t + 1, 1 - b)
            process(t, b)
        return 0

    lax.fori_loop(0, (NCHUNK - 1) // 2, pair, 0)
    process(NCHUNK - 1, 0)
    for b in range(2):
        basef = pl.multiple_of(wid * EPW + (NCHUNK - 2 + b) * C, 8)
        pltpu.make_async_copy(vbuf[b], vec_hbm.at[pl.ds(basef, C)],
                              sw[b]).wait()


def _sc_vec(pos_flat, src, dst):
    return pl.kernel(
        _sc_vec_body,
        out_type=jax.ShapeDtypeStruct((E, POSW), f32),
        mesh=_mesh,
        compiler_params=_sc_params,
        scratch_types=[
            pltpu.VMEM((N * 4,), f32),
            pltpu.VMEM((C,), jnp.int32),
            pltpu.VMEM((C,), jnp.int32),
            pltpu.VMEM((C,), jnp.int32),
            pltpu.VMEM((C,), jnp.int32),
            pltpu.VMEM((C, POSW), f32),
            pltpu.VMEM((C, POSW), f32),
            pltpu.SemaphoreType.DMA,
            pltpu.SemaphoreType.DMA,
            pltpu.SemaphoreType.DMA,
            pltpu.SemaphoreType.DMA,
            pltpu.SemaphoreType.DMA,
            pltpu.SemaphoreType.DMA,
        ],
    )(pos_flat, src, dst)


def _sc_edge_fwd_body(h_hbm, r_hbm, src_hbm, dst_hbm, agg_hbm,
                      is0, is1, is2, is3, id0, id1, id2, id3,
                      rows0, rows1, rbuf0, rbuf1, agg_sh,
                      si0, si1, si2, si3, sj0, sj1, sj2, sj3,
                      sg0, sg1, sr0, sr1, ss0, ss1):
    cid = lax.axis_index("c")
    sid = lax.axis_index("s")
    wid = sid * NC + cid
    idx_s = (is0, is1, is2, is3)
    idx_d = (id0, id1, id2, id3)
    rows = (rows0, rows1)
    rbuf = (rbuf0, rbuf1)
    si = (si0, si1, si2, si3)
    sj = (sj0, sj1, sj2, sj3)
    sg = (sg0, sg1)
    sr = (sr0, sr1)
    ss = (ss0, ss1)
    _zero_shared(agg_sh, rbuf0, sid)
    plsc.subcore_barrier()

    def base_of(t):
        return pl.multiple_of(wid * EPW + t * C, 8)

    def idx_load(t, q):
        base = base_of(t)
        pltpu.async_copy(src_hbm.at[pl.ds(base, C)], idx_s[q], si[q])
        pltpu.async_copy(dst_hbm.at[pl.ds(base, C)], idx_d[q], sj[q])

    def idx_wait(t, q):
        base = base_of(t)
        pltpu.make_async_copy(src_hbm.at[pl.ds(base, C)], idx_s[q],
                              si[q]).wait()
        pltpu.make_async_copy(dst_hbm.at[pl.ds(base, C)], idx_d[q],
                              sj[q]).wait()

    def gather_issue(t, b, q):
        pltpu.async_copy(h_hbm.at[idx_s[q]], rows[b], sg[b])
        pltpu.async_copy(r_hbm.at[pl.ds(base_of(t), C)], rbuf[b], sr[b])

    def scatter_drain(b, q):
        pltpu.make_async_copy(rbuf[b], agg_sh.at[idx_d[q]], ss[b]).wait()

    def compute(t, b, q):
        pltpu.make_async_copy(h_hbm.at[idx_s[q]], rows[b], sg[b]).wait()
        pltpu.make_async_copy(r_hbm.at[pl.ds(base_of(t), C)], rbuf[b],
                              sr[b]).wait()

        def row(i2, _):
            for k in range(2):
                i = i2 * 2 + k
                for j in range(H // 16):
                    sl = pl.ds(j * 16, 16)
                    rbuf[b][i, sl] = rows[b][i, sl] * rbuf[b][i, sl]
            return 0

        lax.fori_loop(0, C // 2, row, 0)
        pltpu.async_copy(rbuf[b], agg_sh.at[idx_d[q]], ss[b], add=True)

    idx_load(0, 0)
    idx_load(1, 1)
    idx_wait(0, 0)
    gather_issue(0, 0, 0)

    def quad(t4, _):
        for k in range(4):
            t = 4 * t4 + k
            b = k % 2
            q = k
            qn = (k + 1) % 4
            qn2 = (k + 2) % 4
            if k == 3:
                @pl.when(t4 < (NCHUNK // 4) - 1)
                def _():
                    idx_load(4 * t4 + k + 2, qn2)
            else:
                idx_load(t + 2, qn2)
            idx_wait(t + 1, qn)
            if k == 0:
                @pl.when(t4 > 0)
                def _():
                    scatter_drain(1 - b, (k - 1) % 4)
            else:
                scatter_drain(1 - b, (k - 1) % 4)
            gather_issue(t + 1, 1 - b, qn)
            compute(t, b, q)
        return 0

    lax.fori_loop(0, NCHUNK // 4, quad, 0)
    scatter_drain(1, 3)
    compute(NCHUNK - 1, 0, 0)
    scatter_drain(0, 0)
    plsc.subcore_barrier()
    _copy_out_shared(agg_sh, agg_hbm, cid, sid)


def _sc_edge_fwd(h, R, src, dst):
    return pl.kernel(
        _sc_edge_fwd_body,
        out_type=jax.ShapeDtypeStruct((NC, N, H), f32),
        mesh=_mesh,
        compiler_params=_sc_params,
        scratch_types=(
            [pltpu.VMEM((C,), jnp.int32)] * 8
            + [pltpu.VMEM((C, H), f32)] * 4
            + [pltpu.VMEM_SHARED((N, H), f32)]
            + [pltpu.SemaphoreType.DMA] * 14
        ),
    )(h, R, src, dst)


def _sc_edge_bwd1_body(dagg_hbm, h_hbm, r_hbm, src_hbm, dst_hbm, dr_hbm,
                       dh_hbm, idx_s0, idx_s1, idx_d0, idx_d1, idx_st, idx_dt,
                       drows0, drows1, hrows0, hrows1, rbuf0, rbuf1, dh_sh,
                       sa0, sa1, sh0, sh1, sr0, sr1, sw0, sw1, ss0, ss1):
    cid = lax.axis_index("c")
    sid = lax.axis_index("s")
    wid = sid * NC + cid
    idx_s = (idx_s0, idx_s1)
    idx_d = (idx_d0, idx_d1)
    drows = (drows0, drows1)
    hrows = (hrows0, hrows1)
    rbuf = (rbuf0, rbuf1)
    sa = (sa0, sa1)
    sh = (sh0, sh1)
    sr = (sr0, sr1)
    sw = (sw0, sw1)
    ss = (ss0, ss1)
    _zero_shared(dh_sh, rbuf0, sid)
    plsc.subcore_barrier()
    NCH1 = EPW // C1

    def load(t, b):
        @pl.when(t >= 2)
        def _():
            base0 = pl.multiple_of(wid * EPW + (t - 2) * C1, 8)
            pltpu.make_async_copy(hrows[b], dr_hbm.at[pl.ds(base0, C1)],
                                  sw[b]).wait()
            pltpu.make_async_copy(rbuf[b], dh_sh.at[idx_s[b]], ss[b]).wait()

        base = pl.multiple_of(wid * EPW + t * C1, 8)
        pltpu.sync_copy(src_hbm.at[pl.ds(base, C1)], idx_s[b])
        pltpu.sync_copy(dst_hbm.at[pl.ds(base, C1)], idx_d[b])
        pltpu.async_copy(dagg_hbm.at[idx_d[b]], drows[b], sa[b])
        pltpu.async_copy(h_hbm.at[idx_s[b]], hrows[b], sh[b])
        pltpu.async_copy(r_hbm.at[pl.ds(base, C1)], rbuf[b], sr[b])

    def process(t, b):
        base = pl.multiple_of(wid * EPW + t * C1, 8)
        pltpu.make_async_copy(dagg_hbm.at[idx_d[b]], drows[b], sa[b]).wait()
        pltpu.make_async_copy(h_hbm.at[idx_s[b]], hrows[b], sh[b]).wait()
        pltpu.make_async_copy(r_hbm.at[pl.ds(base, C1)], rbuf[b], sr[b]).wait()

        def row(i2, _):
            for k in range(2):
                i = i2 * 2 + k
                for j in range(H // 16):
                    sl = pl.ds(j * 16, 16)
                    dm = drows[b][i, sl]
                    hrows[b][i, sl] = dm * hrows[b][i, sl]
                    rbuf[b][i, sl] = dm * rbuf[b][i, sl]
            return 0

        lax.fori_loop(0, C1 // 2, row, 0)
        pltpu.async_copy(hrows[b], dr_hbm.at[pl.ds(base, C1)], sw[b])
        pltpu.async_copy(rbuf[b], dh_sh.at[idx_s[b]], ss[b], add=True)

    load(0, 0)

    def pair(t2, _):
        load(2 * t2 + 1, 1)
        process(2 * t2, 0)

        @pl.when(t2 < (NCH1 // 2) - 1)
        def _():
            load(2 * t2 + 2, 0)

        process(2 * t2 + 1, 1)
        return 0

    lax.fori_loop(0, NCH1 // 2, pair, 0)
    for b in range(2):
        basef = pl.multiple_of(wid * EPW + (NCH1 - 2 + b) * C1, 8)
        pltpu.make_async_copy(hrows[b], dr_hbm.at[pl.ds(basef, C1)],
                              sw[b]).wait()
        pltpu.make_async_copy(rbuf[b], dh_sh.at[idx_s[b]], ss[b]).wait()

    base = pl.multiple_of(wid * EPW + NCH1 * C1, 8)
    pltpu.sync_copy(src_hbm.at[pl.ds(base, CT)], idx_st)
    pltpu.sync_copy(dst_hbm.at[pl.ds(base, CT)], idx_dt)
    pltpu.async_copy(dagg_hbm.at[idx_dt], drows0.at[pl.ds(0, CT)], sa0).wait()
    pltpu.async_copy(h_hbm.at[idx_st], hrows0.at[pl.ds(0, CT)], sh0).wait()
    pltpu.async_copy(r_hbm.at[pl.ds(base, CT)], rbuf0.at[pl.ds(0, CT)],
                     sr0).wait()

    def trow(i, _):
        for j in range(H // 16):
            sl = pl.ds(j * 16, 16)
            dm = drows0[i, sl]
            hrows0[i, sl] = dm * hrows0[i, sl]
            rbuf0[i, sl] = dm * rbuf0[i, sl]
        return 0

    lax.fori_loop(0, CT, trow, 0)
    pltpu.sync_copy(hrows0.at[pl.ds(0, CT)], dr_hbm.at[pl.ds(base, CT)])
    pltpu.sync_copy(rbuf0.at[pl.ds(0, CT)], dh_sh.at[idx_st], add=True)

    plsc.subcore_barrier()
    _copy_out_shared(dh_sh, dh_hbm, cid, sid)


def _sc_edge_bwd1(dagg, h, R, src, dst):
    return pl.kernel(
        _sc_edge_bwd1_body,
        out_type=(jax.ShapeDtypeStruct((E, H), f32),
                  jax.ShapeDtypeStruct((NC, N, H), f32)),
        mesh=_mesh,
        compiler_params=_sc_params,
        scratch_types=[
            pltpu.VMEM((C1,), jnp.int32),
            pltpu.VMEM((C1,), jnp.int32),
            pltpu.VMEM((C1,), jnp.int32),
            pltpu.VMEM((C1,), jnp.int32),
            pltpu.VMEM((CT,), jnp.int32),
            pltpu.VMEM((CT,), jnp.int32),
            pltpu.VMEM((C1, H), f32),
            pltpu.VMEM((C1, H), f32),
            pltpu.VMEM((C1, H), f32),
            pltpu.VMEM((C1, H), f32),
            pltpu.VMEM((C1, H), f32),
            pltpu.VMEM((C1, H), f32),
            pltpu.VMEM_SHARED((N, H), f32),
            pltpu.SemaphoreType.DMA,
            pltpu.SemaphoreType.DMA,
            pltpu.SemaphoreType.DMA,
            pltpu.SemaphoreType.DMA,
            pltpu.SemaphoreType.DMA,
            pltpu.SemaphoreType.DMA,
            pltpu.SemaphoreType.DMA,
            pltpu.SemaphoreType.DMA,
            pltpu.SemaphoreType.DMA,
            pltpu.SemaphoreType.DMA,
        ],
    )(dagg, h, R, src, dst)


def _sc_edge_bwd0_body(dagg_hbm, h_hbm, src_hbm, dst_hbm, dr_hbm,
                       is0, is1, is2, is3, id0, id1, id2, id3,
                       drows0, drows1, hrows0, hrows1,
                       si0, si1, si2, si3, sj0, sj1, sj2, sj3,
                       sa0, sa1, sh0, sh1, sw0, sw1):
    wid = _wid()
    idx_s = (is0, is1, is2, is3)
    idx_d = (id0, id1, id2, id3)
    drows = (drows0, drows1)
    hrows = (hrows0, hrows1)
    si = (si0, si1, si2, si3)
    sj = (sj0, sj1, sj2, sj3)
    sa = (sa0, sa1)
    sh = (sh0, sh1)
    sw = (sw0, sw1)

    def base_of(t):
        return pl.multiple_of(wid * EPW + t * C, 8)

    def idx_load(t, q):
        base = base_of(t)
        pltpu.async_copy(src_hbm.at[pl.ds(base, C)], idx_s[q], si[q])
        pltpu.async_copy(dst_hbm.at[pl.ds(base, C)], idx_d[q], sj[q])

    def idx_wait(t, q):
        base = base_of(t)
        pltpu.make_async_copy(src_hbm.at[pl.ds(base, C)], idx_s[q],
                              si[q]).wait()
        pltpu.make_async_copy(dst_hbm.at[pl.ds(base, C)], idx_d[q],
                              sj[q]).wait()

    def gather_issue(t, b, q):
        pltpu.async_copy(dagg_hbm.at[idx_d[q]], drows[b], sa[b])
        pltpu.async_copy(h_hbm.at[idx_s[q]], hrows[b], sh[b])

    def write_drain(t, b):
        pltpu.make_async_copy(hrows[b], dr_hbm.at[pl.ds(base_of(t), C)],
                              sw[b]).wait()

    def compute(t, b, q):
        pltpu.make_async_copy(dagg_hbm.at[idx_d[q]], drows[b], sa[b]).wait()
        pltpu.make_async_copy(h_hbm.at[idx_s[q]], hrows[b], sh[b]).wait()

        def row(i2, _):
            for k in range(2):
                i = i2 * 2 + k
                for j in range(H // 16):
                    sl = pl.ds(j * 16, 16)
                    hrows[b][i, sl] = drows[b][i, sl] * hrows[b][i, sl]
            return 0

        lax.fori_loop(0, C // 2, row, 0)
        pltpu.async_copy(hrows[b], dr_hbm.at[pl.ds(base_of(t), C)], sw[b])

    idx_load(0, 0)
    idx_load(1, 1)
    idx_wait(0, 0)
    gather_issue(0, 0, 0)

    def quad(t4, _):
        for k in range(4):
            t = 4 * t4 + k
            b = k % 2
            q = k
            qn = (k + 1) % 4
            qn2 = (k + 2) % 4
            if k == 3:
                @pl.when(t4 < (NCHUNK // 4) - 1)
                def _():
                    idx_load(4 * t4 + k + 2, qn2)
            else:
                idx_load(t + 2, qn2)
            idx_wait(t + 1, qn)
            if k == 0:
                @pl.when(t4 > 0)
                def _():
                    write_drain(4 * t4 - 1, 1 - b)
            else:
                write_drain(t - 1, 1 - b)
            gather_issue(t + 1, 1 - b, qn)
            compute(t, b, q)
        return 0

    lax.fori_loop(0, NCHUNK // 4, quad, 0)
    write_drain(NCHUNK - 2, 1)
    compute(NCHUNK - 1, 0, 0)
    write_drain(NCHUNK - 1, 0)


def _sc_edge_bwd0(dagg, h, src, dst):
    return pl.kernel(
        _sc_edge_bwd0_body,
        out_type=jax.ShapeDtypeStruct((E, H), f32),
        mesh=_mesh,
        compiler_params=_sc_params,
        scratch_types=(
            [pltpu.VMEM((C,), jnp.int32)] * 8
            + [pltpu.VMEM((C, H), f32)] * 4
            + [pltpu.SemaphoreType.DMA] * 14
        ),
    )(dagg, h, src, dst)


def _sc_forces_body(dv_hbm, src_hbm, dst_hbm, fp_hbm, idx_s, idx_d, dvb,
                    acc, sv):
    wid = _wid()
    zz = jnp.zeros((16,), f32)

    def zrow(k, _):
        acc[pl.ds(k * 16, 16)] = zz
        return 0

    lax.fori_loop(0, (N * 4) // 16, zrow, 0)

    def do_chunk(q):
        base = pl.multiple_of(q * CF, 8)
        pltpu.sync_copy(src_hbm.at[pl.ds(base, CF)], idx_s)
        pltpu.sync_copy(dst_hbm.at[pl.ds(base, CF)], idx_d)
        pltpu.async_copy(dv_hbm.at[:, pl.ds(base, CF)], dvb, sv).wait()

        def group(g, _):
            s16 = idx_s[pl.ds(g * 16, 16)] * 4
            d16 = idx_d[pl.ds(g * 16, 16)] * 4
            for c in range(3):
                vals = dvb[c, pl.ds(g * 16, 16)]
                plsc.addupdate_scatter(acc, [s16 + c], vals)
                plsc.addupdate_scatter(acc, [d16 + c], -vals)
            return 0

        lax.fori_loop(0, CF // 16, group, 0)

    def chunk(t, _):
        do_chunk(wid + NW * t)
        return 0

    lax.fori_loop(0, (E // CF) // NW, chunk, 0)

    @pl.when(wid < (E // CF) % NW)
    def _():
        do_chunk(((E // CF) // NW) * NW + wid)

    pltpu.sync_copy(acc, fp_hbm.at[pl.ds(wid * (N * 4), N * 4)])


def _sc_forces(dvec_t, src, dst):
    return pl.kernel(
        _sc_forces_body,
        out_type=jax.ShapeDtypeStruct((NW * N * 4,), f32),
        mesh=_mesh,
        compiler_params=_sc_params,
        scratch_types=[
            pltpu.VMEM((CF,), jnp.int32),
            pltpu.VMEM((CF,), jnp.int32),
            pltpu.VMEM((4, CF), f32),
            pltpu.VMEM((N * 4,), f32),
            pltpu.SemaphoreType.DMA,
        ],
    )(dvec_t, src, dst)


BN = 1000
BE = 1280


def _silu_prime(z):
    sg = jax.nn.sigmoid(z)
    return sg * (1.0 + z * (1.0 - sg))


def _embed_body(at_ref, w_ref, o_ref):
    t = at_ref[0, 0, :]
    oh = (t[:, None] == lax.broadcasted_iota(jnp.int32, (1, NUM_ELEM), 1)
          ).astype(f32)
    o_ref[...] = jnp.dot(oh, w_ref[...], preferred_element_type=f32)


def _embed(atom_types, W_emb):
    at3 = atom_types.reshape(N // BN, 1, BN)
    return pl.pallas_call(
        _embed_body,
        grid=(N // BN,),
        in_specs=[
            pl.BlockSpec((1, 1, BN), lambda i: (i, 0, 0)),
            pl.BlockSpec((NUM_ELEM, H), lambda i: (0, 0)),
        ],
        out_specs=pl.BlockSpec((BN, H), lambda i: (i, 0)),
        out_shape=jax.ShapeDtypeStruct((N, H), f32),
    )(at3, W_emb)


def _trig1(r):
    rc = jnp.clip(r, 0.0, RMAX)
    t = (math.pi / RMAX) * rc - (math.pi / 2)
    u = t * t
    s1 = 1.0 + u * (-1.0 / 2 + u * (1.0 / 24 + u * (-1.0 / 720 + u * (
        1.0 / 40320 + u * (-1.0 / 3628800 + u * (1.0 / 479001600))))))
    sp = 1.0 + u * (-1.0 / 6 + u * (1.0 / 120 + u * (-1.0 / 5040 + u * (
        1.0 / 362880 + u * (-1.0 / 39916800)))))
    c1 = -t * sp
    return s1, c1


def _harm_sigma(s1, c1, r):
    kn = math.pi / RMAX
    x = kn * jnp.clip(r, 0.0, RMAX)
    sigma1 = kn * jnp.where(x < 0.8, _sincp(x * x), s1 / jnp.maximum(x, 0.5))
    rs1 = r * s1
    sig, cn = sigma1, c1
    sigs, ccs = [sig], [cn]
    for _ in range(NB - 1):
        sig, cn = sig * c1 + cn * sigma1, cn * c1 - sig * rs1
        sigs.append(sig)
        ccs.append(cn)
    return sigs, ccs


def _stack8(cols):
    return jnp.concatenate([x[:, None] for x in cols], axis=1)


def _sincp(w):
    return 1.0 + w * (-1.0 / 6 + w * (1.0 / 120 + w * (-1.0 / 5040 + w * (
        1.0 / 362880))))


def _radial_fwd_body(vec_ref, w1c_ref, w2bd_ref, rbt_ref, vt_ref,
                     r0_ref, r1_ref):
    vt16 = jnp.transpose(vec_ref[...])
    vt = vt16[:4, :]
    s = vt[0] * vt[0] + vt[1] * vt[1] + vt[2] * vt[2] + 1e-12
    r = jnp.sqrt(s)
    s1, c1 = _trig1(r)
    sigs, _ = _harm_sigma(s1, c1, r)
    env = 0.5 * (c1 + 1.0)
    coef = math.sqrt(2.0 / RMAX) * env
    rbt = jnp.concatenate([(coef * sg)[None, :] for sg in sigs], axis=0)
    rbt_ref[...] = rbt
    vt_ref[...] = vt
    aq = lax.dot_general(rbt, w1c_ref[...], (((0,), (0,)), ((), ())),
                         preferred_element_type=f32)
    s01 = aq * jax.nn.sigmoid(aq)
    r01 = jnp.dot(s01, w2bd_ref[...], preferred_element_type=f32)
    r0_ref[...] = r01[:, :H]
    r1_ref[...] = r01[:, H:]


def _radial_fwd(vec16, W1cat, W2bd):
    return pl.pallas_call(
        _radial_fwd_body,
        grid=(E // BE,),
        in_specs=[
            pl.BlockSpec((BE, POSW), lambda i: (i, 0)),
            pl.BlockSpec((NB, 2 * RH), lambda i: (0, 0)),
            pl.BlockSpec((2 * RH, 2 * H), lambda i: (0, 0)),
        ],
        out_specs=[
            pl.BlockSpec((NB, BE), lambda i: (0, i)),
            pl.BlockSpec((4, BE), lambda i: (0, i)),
            pl.BlockSpec((BE, H), lambda i: (i, 0)),
            pl.BlockSpec((BE, H), lambda i: (i, 0)),
        ],
        out_shape=[
            jax.ShapeDtypeStruct((NB, E), f32),
            jax.ShapeDtypeStruct((4, E), f32),
            jax.ShapeDtypeStruct((E, H), f32),
            jax.ShapeDtypeStruct((E, H), f32),
        ],
    )(vec16, W1cat, W2bd)


def _node_fwd_body(aggp_ref, wu_ref, z_ref, h_ref):
    a = aggp_ref[0] + aggp_ref[1]
    z = jnp.dot(a, wu_ref[...], preferred_element_type=f32)
    z_ref[...] = z
    h_ref[...] = z * jax.nn.sigmoid(z)


def _node_fwd(aggp, Wu):
    return pl.pallas_call(
        _node_fwd_body,
        grid=(N // BN,),
        in_specs=[
            pl.BlockSpec((NC, BN, H), lambda i: (0, i, 0)),
            pl.BlockSpec((H, H), lambda i: (0, 0)),
        ],
        out_specs=[
            pl.BlockSpec((BN, H), lambda i: (i, 0)),
            pl.BlockSpec((BN, H), lambda i: (i, 0)),
        ],
        out_shape=[
            jax.ShapeDtypeStruct((N, H), f32),
            jax.ShapeDtypeStruct((N, H), f32),
        ],
    )(aggp, Wu)


def _top_bwd_body(z_ref, h2_ref, wuT_ref, wo_ref, dagg_ref, e_ref):
    i = pl.program_id(0)
    z = z_ref[...]
    wo = wo_ref[...]
    dz = wo * _silu_prime(z)
    dagg_ref[...] = jnp.dot(dz, wuT_ref[...], preferred_element_type=f32)
    part = jnp.sum(h2_ref[...] * wo)

    @pl.when(i == 0)
    def _():
        e_ref[0, 0] = 0.0

    e_ref[0, 0] += part


def _top_bwd(z1, h2, Wu_1T, wo2d):
    return pl.pallas_call(
        _top_bwd_body,
        grid=(N // BN,),
        in_specs=[
            pl.BlockSpec((BN, H), lambda i: (i, 0)),
            pl.BlockSpec((BN, H), lambda i: (i, 0)),
            pl.BlockSpec((H, H), lambda i: (0, 0)),
            pl.BlockSpec((1, H), lambda i: (0, 0)),
        ],
        out_specs=[
            pl.BlockSpec((BN, H), lambda i: (i, 0)),
            pl.BlockSpec(memory_space=pltpu.SMEM),
        ],
        out_shape=[
            jax.ShapeDtypeStruct((N, H), f32),
            jax.ShapeDtypeStruct((1, 1), f32),
        ],
    )(z1, h2, Wu_1T, wo2d)


def _mid_bwd_body(dhp_ref, z_ref, wuT_ref, dagg_ref):
    dh = dhp_ref[0] + dhp_ref[1]
    z = z_ref[...]
    dagg_ref[...] = jnp.dot(dh * _silu_prime(z), wuT_ref[...],
                            preferred_element_type=f32)


def _mid_bwd(dhp, z0, Wu_0T):
    return pl.pallas_call(
        _mid_bwd_body,
        grid=(N // BN,),
        in_specs=[
            pl.BlockSpec((NC, BN, H), lambda i: (0, i, 0)),
            pl.BlockSpec((BN, H), lambda i: (i, 0)),
            pl.BlockSpec((H, H), lambda i: (0, 0)),
        ],
        out_specs=pl.BlockSpec((BN, H), lambda i: (i, 0)),
        out_shape=jax.ShapeDtypeStruct((N, H), f32),
    )(dhp, z0, Wu_0T)


def _radial_bwd_body(dr0_ref, dr1_ref, rbt_ref, vt_ref, w1c_ref, w2bdT_ref,
                     dv_ref):
    vt = vt_ref[...]
    s = vt[0] * vt[0] + vt[1] * vt[1] + vt[2] * vt[2] + 1e-12
    r = jnp.sqrt(s)
    s1, c1 = _trig1(r)
    sigs, ccs = _harm_sigma(s1, c1, r)
    env = 0.5 * (c1 + 1.0)
    rbt = rbt_ref[...]
    aq = lax.dot_general(rbt, w1c_ref[...], (((0,), (0,)), ((), ())),
                         preferred_element_type=f32)
    dr01 = jnp.concatenate([dr0_ref[...], dr1_ref[...]], axis=1)
    ds01 = jnp.dot(dr01, w2bdT_ref[...], preferred_element_type=f32)
    da01 = ds01 * _silu_prime(aq)
    drbt = lax.dot_general(w1c_ref[...], da01, (((1,), (1,)), ((), ())),
                           preferred_element_type=f32)
    c0 = math.sqrt(2.0 / RMAX)
    rinv = 1.0 / r
    kn = math.pi / RMAX
    envp = jnp.where(r < RMAX, -0.5 * kn * s1, 0.0)
    er = env * rinv
    wmat = jnp.concatenate(
        [(c0 * (er * ((n + 1) * kn * cn - sg) + envp * sg))[None, :]
         for n, (sg, cn) in enumerate(zip(sigs, ccs))], axis=0)
    dr = jnp.sum(drbt * wmat, axis=0)
    dv_ref[...] = (dr * rinv)[None, :] * vt


def _radial_bwd(dR0, dR1, rbt, vec_t, W1cat, W2bdT):
    return pl.pallas_call(
        _radial_bwd_body,
        grid=(E // BE,),
        in_specs=[
            pl.BlockSpec((BE, H), lambda i: (i, 0)),
            pl.BlockSpec((BE, H), lambda i: (i, 0)),
            pl.BlockSpec((NB, BE), lambda i: (0, i)),
            pl.BlockSpec((4, BE), lambda i: (0, i)),
            pl.BlockSpec((NB, 2 * RH), lambda i: (0, 0)),
            pl.BlockSpec((2 * H, 2 * RH), lambda i: (0, 0)),
        ],
        out_specs=pl.BlockSpec((4, BE), lambda i: (0, i)),
        out_shape=jax.ShapeDtypeStruct((4, E), f32),
    )(dR0, dR1, rbt, vec_t, W1cat, W2bdT)


FRB = 4096


def _force_reduce_body(fp_ref, o_ref):
    o_ref[...] = jnp.sum(fp_ref[...], axis=0)


def _force_reduce(fp):
    nblk = (N * 4 + FRB - 1) // FRB
    return pl.pallas_call(
        _force_reduce_body,
        grid=(nblk,),
        in_specs=[pl.BlockSpec((NW, FRB), lambda i: (0, i))],
        out_specs=pl.BlockSpec((FRB,), lambda i: (i,)),
        out_shape=jax.ShapeDtypeStruct((N * 4,), f32),
    )(fp)


def kernel(positions, atom_types, edge_index, batch, W_emb, Wr1_0, Wr2_0,
           Wu_0, Wr1_1, Wr2_1, Wu_1, w_out):
    del batch
    pos_flat = jnp.concatenate(
        [positions, jnp.zeros((N, 1), f32)], axis=1).reshape(N * 4)
    src = edge_index[0]
    dst = edge_index[1]

    W1cat = jnp.concatenate([Wr1_0, Wr1_1], axis=1)
    W2bd = jnp.zeros((2 * RH, 2 * H), f32)
    W2bd = W2bd.at[:RH, :H].set(Wr2_0).at[RH:, H:].set(Wr2_1)
    W2bdT = jnp.zeros((2 * H, 2 * RH), f32)
    W2bdT = W2bdT.at[:H, :RH].set(Wr2_0.T).at[H:, RH:].set(Wr2_1.T)

    h0 = _embed(atom_types, W_emb)
    vec16 = _sc_vec(pos_flat, src, dst)
    rbt, vec_t, R0, R1 = _radial_fwd(vec16, W1cat, W2bd)

    aggp0 = _sc_edge_fwd(h0, R0, src, dst)
    z0, h1 = _node_fwd(aggp0, Wu_0)
    aggp1 = _sc_edge_fwd(h1, R1, src, dst)
    z1, h2 = _node_fwd(aggp1, Wu_1)

    dagg1, e11 = _top_bwd(z1, h2, Wu_1.T, w_out.reshape(1, H))
    dR1, dhp1 = _sc_edge_bwd1(dagg1, h1, R1, src, dst)
    dagg0 = _mid_bwd(dhp1, z0, Wu_0.T)
    dR0 = _sc_edge_bwd0(dagg0, h0, src, dst)

    dvec_t = _radial_bwd(dR0, dR1, rbt, vec_t, W1cat, W2bdT)
    fp = _sc_forces(dvec_t, src, dst).reshape(NW, N * 4)
    forces4 = _force_reduce(fp)

    energy = e11.reshape(1)
    forces = forces4.reshape(N, 4)[:, :3]
    return energy, forces
